# Initial kernel scaffold; baseline (speedup 1.0000x reference)
import jax
import jax.numpy as jnp
from jax.experimental import pallas as pl


def kernel(x, edge_index, edge_attr, batch, params):
    raise NotImplementedError("write your pallas kernel here")


# speedup vs baseline: 3.0074x; 3.0074x over previous
import functools

import jax
import jax.numpy as jnp
from jax import lax
from jax.experimental import pallas as pl
from jax.experimental.pallas import tpu as pltpu
from jax.experimental.pallas import tpu_sc as plsc

_N = 50000
_E = 800000
_EMB = 64
_G = 128
_NL = 5
_H = 32
_NC = 2
_NS = 16
_CH = 128
_NCHUNK = _E // _CH
_ZB = 3128
_ZT = _N - (_NS - 1) * _ZB

_F32 = jnp.float32


def _f32(*shape):
    return jax.ShapeDtypeStruct(shape, _F32)


def _matmul_bias_kernel(x_ref, w_ref, b_ref, o_ref):
    o_ref[...] = (
        jnp.dot(x_ref[...], w_ref[...].T, preferred_element_type=_F32)
        + b_ref[...]
    )


def _node_embed(x, w, b):
    nb = 1000
    return pl.pallas_call(
        _matmul_bias_kernel,
        grid=(_N // nb,),
        in_specs=[
            pl.BlockSpec((nb, 40), lambda i: (i, 0)),
            pl.BlockSpec((_EMB, 40), lambda i: (0, 0)),
            pl.BlockSpec((1, _EMB), lambda i: (0, 0)),
        ],
        out_specs=pl.BlockSpec((nb, _EMB), lambda i: (i, 0)),
        out_shape=_f32(_N, _EMB),
    )(x, w, b.reshape(1, _EMB))


def _edge_embed_kernel(a_ref, w_ref, b_ref, o_ref):
    ea = (
        jnp.dot(a_ref[...], w_ref[...].T, preferred_element_type=_F32)
        + b_ref[...]
    )
    o_ref[0] = ea[:, :_H]
    o_ref[1] = ea[:, _H:]


def _edge_embed(edge_attr, w, b):
    eb = 2000
    return pl.pallas_call(
        _edge_embed_kernel,
        grid=(_E // eb,),
        in_specs=[
            pl.BlockSpec((eb, 10), lambda i: (i, 0)),
            pl.BlockSpec((_EMB, 10), lambda i: (0, 0)),
            pl.BlockSpec((1, _EMB), lambda i: (0, 0)),
        ],
        out_specs=pl.BlockSpec((2, eb, _H), lambda i: (0, i, 0)),
        out_shape=_f32(2, _E, _H),
    )(edge_attr, w, b.reshape(1, _EMB))


def _degpost_kernel(dp_ref, dis_ref, r_ref, disw_ref):
    d = dp_ref[0, :, 0:1] + dp_ref[1, :, 0:1] + 1.0
    dis = lax.rsqrt(d)
    dis_ref[...] = dis
    r_ref[...] = 1.0 / d
    disw_ref[...] = jnp.broadcast_to(dis, (dis.shape[0], _H))


def _degpost(degpart):
    nb = 1000
    return pl.pallas_call(
        _degpost_kernel,
        grid=(_N // nb,),
        in_specs=[pl.BlockSpec((2, nb, 8), lambda i: (0, i, 0))],
        out_specs=[
            pl.BlockSpec((nb, 1), lambda i: (i, 0)),
            pl.BlockSpec((nb, 1), lambda i: (i, 0)),
            pl.BlockSpec((nb, _H), lambda i: (i, 0)),
        ],
        out_shape=[_f32(_N, 1), _f32(_N, 1), _f32(_N, _H)],
    )(degpart)


def _pre_kernel(h_ref, dis_ref, r_ref, w_ref, b_ref, root_ref, g2_ref, st_ref):
    hp = (
        jnp.dot(h_ref[...], w_ref[...].T, preferred_element_type=_F32)
        + b_ref[...]
    )
    g = hp * dis_ref[...]
    g2_ref[0] = g[:, :_H]
    g2_ref[1] = g[:, _H:]
    st_ref[...] = jnp.maximum(hp + root_ref[...], 0.0) * r_ref[...]


def _layer_pre(h, dis, r, w, b, root):
    nb = 1000
    return pl.pallas_call(
        _pre_kernel,
        grid=(_N // nb,),
        in_specs=[
            pl.BlockSpec((nb, _EMB), lambda i: (i, 0)),
            pl.BlockSpec((nb, 1), lambda i: (i, 0)),
            pl.BlockSpec((nb, 1), lambda i: (i, 0)),
            pl.BlockSpec((_EMB, _EMB), lambda i: (0, 0)),
            pl.BlockSpec((1, _EMB), lambda i: (0, 0)),
            pl.BlockSpec((1, _EMB), lambda i: (0, 0)),
        ],
        out_specs=[
            pl.BlockSpec((2, nb, _H), lambda i: (0, i, 0)),
            pl.BlockSpec((nb, _EMB), lambda i: (i, 0)),
        ],
        out_shape=[_f32(2, _N, _H), _f32(_N, _EMB)],
    )(h, dis, r, w, b.reshape(1, _EMB), root)


def _post_kernel(agg_ref, dis_ref, st_ref, hin_ref, z_ref, stats_ref):
    aggf = jnp.concatenate([agg_ref[0], agg_ref[1]], axis=1)
    z = hin_ref[...] + dis_ref[...] * aggf + st_ref[...]
    z_ref[...] = z
    blockstats = jnp.concatenate(
        [jnp.sum(z, axis=0, keepdims=True),
         jnp.sum(z * z, axis=0, keepdims=True)], axis=0)

    @pl.when(pl.program_id(0) == 0)
    def _():
        stats_ref[...] = blockstats

    @pl.when(pl.program_id(0) != 0)
    def _():
        stats_ref[...] = stats_ref[...] + blockstats


def _layer_post(agg, dis, st, hin):
    nb = 1000
    return pl.pallas_call(
        _post_kernel,
        grid=(_N // nb,),
        in_specs=[
            pl.BlockSpec((2, nb, _H), lambda i: (0, i, 0)),
            pl.BlockSpec((nb, 1), lambda i: (i, 0)),
            pl.BlockSpec((nb, _EMB), lambda i: (i, 0)),
            pl.BlockSpec((nb, _EMB), lambda i: (i, 0)),
        ],
        out_specs=[
            pl.BlockSpec((nb, _EMB), lambda i: (i, 0)),
            pl.BlockSpec((2, _EMB), lambda i: (0, 0)),
        ],
        out_shape=[_f32(_N, _EMB), _f32(2, _EMB)],
    )(agg, dis, st, hin)


def _bn_kernel(z_ref, stats_ref, g_ref, b_ref, o_ref, *, relu):
    mu = stats_ref[0:1, :] * (1.0 / _N)
    var = stats_ref[1:2, :] * (1.0 / _N) - mu * mu
    inv = lax.rsqrt(var + 1e-5)
    h = (z_ref[...] - mu) * inv * g_ref[...] + b_ref[...]
    if relu:
        h = jnp.maximum(h, 0.0)
    o_ref[...] = h


def _bn_apply(z, stats, g, b, relu):
    nb = 1000
    return pl.pallas_call(
        functools.partial(_bn_kernel, relu=relu),
        grid=(_N // nb,),
        in_specs=[
            pl.BlockSpec((nb, _EMB), lambda i: (i, 0)),
            pl.BlockSpec((2, _EMB), lambda i: (0, 0)),
            pl.BlockSpec((1, _EMB), lambda i: (0, 0)),
            pl.BlockSpec((1, _EMB), lambda i: (0, 0)),
        ],
        out_specs=pl.BlockSpec((nb, _EMB), lambda i: (i, 0)),
        out_shape=_f32(_N, _EMB),
    )(z, stats, g.reshape(1, _EMB), b.reshape(1, _EMB))


def _pool_kernel(h_ref, b3_ref, sums_ref, cnt_ref):
    ids = b3_ref[0, 0, :]
    io = lax.broadcasted_iota(jnp.int32, (_G, ids.shape[0]), 0)
    oh = (io == ids[None, :]).astype(_F32)
    ps = jnp.dot(oh, h_ref[...], preferred_element_type=_F32)
    pc = jnp.sum(oh, axis=1, keepdims=True)

    @pl.when(pl.program_id(0) == 0)
    def _():
        sums_ref[...] = ps
        cnt_ref[...] = pc

    @pl.when(pl.program_id(0) != 0)
    def _():
        sums_ref[...] = sums_ref[...] + ps
        cnt_ref[...] = cnt_ref[...] + pc


def _pool(h, batch):
    nb = 1000
    batch3 = batch.reshape(_N // nb, 1, nb)
    return pl.pallas_call(
        _pool_kernel,
        grid=(_N // nb,),
        in_specs=[
            pl.BlockSpec((nb, _EMB), lambda i: (i, 0)),
            pl.BlockSpec((1, 1, nb), lambda i: (i, 0, 0)),
        ],
        out_specs=[
            pl.BlockSpec((_G, _EMB), lambda i: (0, 0)),
            pl.BlockSpec((_G, 1), lambda i: (0, 0)),
        ],
        out_shape=[_f32(_G, _EMB), _f32(_G, 1)],
    )(h, batch3)


def _mlp_kernel(s_ref, c_ref, w1_ref, b1_ref, w2_ref, b2_ref, w3_ref, b3_ref,
                o_ref):
    hg = s_ref[...] / jnp.maximum(c_ref[...], 1.0)
    z = jnp.maximum(
        jnp.dot(hg, w1_ref[...].T, preferred_element_type=_F32) + b1_ref[...],
        0.0)
    z = jnp.maximum(
        jnp.dot(z, w2_ref[...].T, preferred_element_type=_F32) + b2_ref[...],
        0.0)
    o_ref[...] = (
        jnp.sum(z * w3_ref[...], axis=1, keepdims=True) + b3_ref[...]
    )


def _mlp(sums, cnt, pred):
    (w1, b1), (w2, b2), (w3, b3) = pred
    hh = w1.shape[0]
    return pl.pallas_call(
        _mlp_kernel,
        out_shape=_f32(_G, 1),
    )(sums, cnt, w1, b1.reshape(1, hh), w2, b2.reshape(1, hh),
      w3, b3.reshape(1, 1))


def _sc_mesh():
    return plsc.VectorSubcoreMesh(core_axis_name="c", subcore_axis_name="s")


def _sc_params():
    return pltpu.CompilerParams(use_tc_tiling_on_sc=False)


def _acc_init(zeros_hbm, acc_sh, s):
    @pl.when(s < _NS - 1)
    def _():
        pltpu.sync_copy(zeros_hbm, acc_sh.at[pl.ds(s * _ZB, _ZB)])

    @pl.when(s == _NS - 1)
    def _():
        pltpu.sync_copy(zeros_hbm.at[pl.ds(0, _ZT)],
                        acc_sh.at[pl.ds((_NS - 1) * _ZB, _ZT)])


def _acc_flush(acc_sh, out_c, s):
    @pl.when(s < _NS - 1)
    def _():
        pltpu.sync_copy(acc_sh.at[pl.ds(s * _ZB, _ZB)],
                        out_c.at[pl.ds(s * _ZB, _ZB)])

    @pl.when(s == _NS - 1)
    def _():
        pltpu.sync_copy(acc_sh.at[pl.ds((_NS - 1) * _ZB, _ZT)],
                        out_c.at[pl.ds((_NS - 1) * _ZB, _ZT)])


def _deg_sc_body(row_hbm, ones_hbm, zeros_hbm, out_hbm, idx_v, ones_v, acc_sh):
    c = lax.axis_index("c")
    s = lax.axis_index("s")
    _acc_init(zeros_hbm, acc_sh, s)
    pltpu.sync_copy(ones_hbm, ones_v)
    plsc.subcore_barrier()
    my_n = 195 + (s < 5).astype(jnp.int32)
    my_base = c * (_NCHUNK // _NC) + s * 195 + jnp.minimum(s, 5)

    @pl.loop(0, 196)
    def _(k):
        @pl.when(k < my_n)
        def _():
            ch = my_base + k
            pltpu.sync_copy(row_hbm.at[pl.ds(ch * _CH, _CH)], idx_v)
            pltpu.sync_copy(ones_v, acc_sh.at[idx_v], add=True)

    plsc.subcore_barrier()
    _acc_flush(acc_sh, out_hbm.at[c], s)


def _deg_sc(row, ones8, zeros8):
    kfn = pl.kernel(
        _deg_sc_body,
        out_type=_f32(_NC, _N, 8),
        mesh=_sc_mesh(),
        compiler_params=_sc_params(),
        scratch_types=[
            pltpu.VMEM((_CH,), jnp.int32),
            pltpu.VMEM((_CH, 8), _F32),
            pltpu.VMEM_SHARED((_N, 8), _F32),
        ],
    )
    return kfn(row, ones8, zeros8)


def _edge_split(s):
    my_n = 390 + (s < 10).astype(jnp.int32)
    my_base = s * 390 + jnp.minimum(s, 10)
    return my_n, my_base


def _prep_sc_body(disw_hbm, ea_hbm, row_hbm, out_hbm, idx_v, drow_v, ea_v):
    c = lax.axis_index("c")
    s = lax.axis_index("s")
    my_n, my_base = _edge_split(s)

    @pl.loop(0, 391)
    def _(k):
        @pl.when(k < my_n)
        def _():
            ch = my_base + k
            pltpu.sync_copy(row_hbm.at[pl.ds(ch * _CH, _CH)], idx_v)
            pltpu.sync_copy(disw_hbm.at[idx_v], drow_v)
            pltpu.sync_copy(ea_hbm.at[c].at[pl.ds(ch * _CH, _CH)], ea_v)

            @pl.loop(0, _CH)
            def _(b):
                for j in range(0, _H, 16):
                    sl = (b, pl.ds(j, 16))
                    ea_v[sl] = drow_v[sl] * ea_v[sl]

            pltpu.sync_copy(ea_v, out_hbm.at[c].at[pl.ds(ch * _CH, _CH)])


def _prep_sc(disw, ea2, row):
    kfn = pl.kernel(
        _prep_sc_body,
        out_type=_f32(_NC, _E, _H),
        mesh=_sc_mesh(),
        compiler_params=_sc_params(),
        scratch_types=[
            pltpu.VMEM((_CH,), jnp.int32),
            pltpu.VMEM((_CH, _H), _F32),
            pltpu.VMEM((_CH, _H), _F32),
        ],
    )
    return kfn(disw, ea2, row)


def _msg_sc_body(g2_hbm, eaw_hbm, row_hbm, col_hbm, zeros_hbm, out_hbm,
                 idxr_v, idxc_v, g_v, e_v, acc_sh):
    c = lax.axis_index("c")
    s = lax.axis_index("s")
    _acc_init(zeros_hbm, acc_sh, s)
    plsc.subcore_barrier()
    my_n, my_base = _edge_split(s)

    @pl.loop(0, 391)
    def _(k):
        @pl.when(k < my_n)
        def _():
            ch = my_base + k
            pltpu.sync_copy(row_hbm.at[pl.ds(ch * _CH, _CH)], idxr_v)
            pltpu.sync_copy(col_hbm.at[pl.ds(ch * _CH, _CH)], idxc_v)
            pltpu.sync_copy(g2_hbm.at[c].at[idxr_v], g_v)
            pltpu.sync_copy(eaw_hbm.at[c].at[pl.ds(ch * _CH, _CH)], e_v)

            @pl.loop(0, _CH)
            def _(b):
                for j in range(0, _H, 16):
                    sl = (b, pl.ds(j, 16))
                    g_v[sl] = jnp.maximum(g_v[sl] + e_v[sl], 0.0)

            pltpu.sync_copy(g_v, acc_sh.at[idxc_v], add=True)

    plsc.subcore_barrier()
    _acc_flush(acc_sh, out_hbm.at[c], s)


def _msg_sc(g2, eaw2, row, col, zeros_h):
    kfn = pl.kernel(
        _msg_sc_body,
        out_type=_f32(_NC, _N, _H),
        mesh=_sc_mesh(),
        compiler_params=_sc_params(),
        scratch_types=[
            pltpu.VMEM((_CH,), jnp.int32),
            pltpu.VMEM((_CH,), jnp.int32),
            pltpu.VMEM((_CH, _H), _F32),
            pltpu.VMEM((_CH, _H), _F32),
            pltpu.VMEM_SHARED((_N, _H), _F32),
        ],
    )
    return kfn(g2, eaw2, row, col, zeros_h)


def kernel(x, edge_index, edge_attr, batch, params):
    row = edge_index[0]
    col = edge_index[1]

    ones8 = jnp.ones((_CH, 8), _F32)
    zeros8 = jnp.zeros((_ZB, 8), _F32)
    zeros_h = jnp.zeros((_ZB, _H), _F32)

    h = _node_embed(x, params['x_emb_W'], params['x_emb_b'])
    ea2 = _edge_embed(edge_attr, params['edge_emb_W'], params['edge_emb_b'])

    degpart = _deg_sc(row, ones8, zeros8)
    dis, r, disw = _degpost(degpart)

    eaw2 = _prep_sc(disw, ea2, row)

    for l in range(_NL):
        lp = params['layers'][l]
        g2, st = _layer_pre(h, dis, r, lp['lin_W'], lp['lin_b'], lp['root'])
        agg = _msg_sc(g2, eaw2, row, col, zeros_h)
        z, stats = _layer_post(agg, dis, st, h)
        h = _bn_apply(z, stats, lp['bn_g'], lp['bn_b'], relu=(l < _NL - 1))

    sums, cnt = _pool(h, batch)
    return _mlp(sums, cnt, params['pred'])


# speedup vs baseline: 4.5226x; 1.5038x over previous
import functools

import jax
import jax.numpy as jnp
from jax import lax
from jax.experimental import pallas as pl
from jax.experimental.pallas import tpu as pltpu
from jax.experimental.pallas import tpu_sc as plsc

_N = 50000
_E = 800000
_EMB = 64
_G = 128
_NL = 5
_H = 32
_NC = 2
_NS = 16
_CH = 128
_NCHUNK = _E // _CH
_ZB = 3128
_ZT = _N - (_NS - 1) * _ZB

_F32 = jnp.float32


def _f32(*shape):
    return jax.ShapeDtypeStruct(shape, _F32)


def _matmul_bias_kernel(x_ref, w_ref, b_ref, o_ref):
    o_ref[...] = (
        jnp.dot(x_ref[...], w_ref[...].T, preferred_element_type=_F32)
        + b_ref[...]
    )


def _node_embed(x, w, b):
    nb = 1000
    return pl.pallas_call(
        _matmul_bias_kernel,
        grid=(_N // nb,),
        in_specs=[
            pl.BlockSpec((nb, 40), lambda i: (i, 0)),
            pl.BlockSpec((_EMB, 40), lambda i: (0, 0)),
            pl.BlockSpec((1, _EMB), lambda i: (0, 0)),
        ],
        out_specs=pl.BlockSpec((nb, _EMB), lambda i: (i, 0)),
        out_shape=_f32(_N, _EMB),
    )(x, w, b.reshape(1, _EMB))


def _edge_embed_kernel(a_ref, w_ref, b_ref, o_ref):
    ea = (
        jnp.dot(a_ref[...], w_ref[...].T, preferred_element_type=_F32)
        + b_ref[...]
    )
    o_ref[0] = ea[:, :_H]
    o_ref[1] = ea[:, _H:]


def _edge_embed(edge_attr, w, b):
    eb = 2000
    return pl.pallas_call(
        _edge_embed_kernel,
        grid=(_E // eb,),
        in_specs=[
            pl.BlockSpec((eb, 10), lambda i: (i, 0)),
            pl.BlockSpec((_EMB, 10), lambda i: (0, 0)),
            pl.BlockSpec((1, _EMB), lambda i: (0, 0)),
        ],
        out_specs=pl.BlockSpec((2, eb, _H), lambda i: (0, i, 0)),
        out_shape=_f32(2, _E, _H),
    )(edge_attr, w, b.reshape(1, _EMB))


def _degpost_kernel(dp_ref, dis_ref, r_ref, disw_ref):
    d = dp_ref[0, :, 0:1] + dp_ref[1, :, 0:1] + 1.0
    dis = lax.rsqrt(d)
    dis_ref[...] = dis
    r_ref[...] = 1.0 / d
    disw_ref[...] = jnp.broadcast_to(dis, (dis.shape[0], _H))


def _degpost(degpart):
    nb = 1000
    return pl.pallas_call(
        _degpost_kernel,
        grid=(_N // nb,),
        in_specs=[pl.BlockSpec((2, nb, 8), lambda i: (0, i, 0))],
        out_specs=[
            pl.BlockSpec((nb, 1), lambda i: (i, 0)),
            pl.BlockSpec((nb, 1), lambda i: (i, 0)),
            pl.BlockSpec((nb, _H), lambda i: (i, 0)),
        ],
        out_shape=[_f32(_N, 1), _f32(_N, 1), _f32(_N, _H)],
    )(degpart)


def _pre_kernel(h_ref, dis_ref, r_ref, w_ref, b_ref, root_ref, g2_ref, st_ref):
    hp = (
        jnp.dot(h_ref[...], w_ref[...].T, preferred_element_type=_F32)
        + b_ref[...]
    )
    g = hp * dis_ref[...]
    g2_ref[0] = g[:, :_H]
    g2_ref[1] = g[:, _H:]
    st_ref[...] = jnp.maximum(hp + root_ref[...], 0.0) * r_ref[...]


def _layer_pre(h, dis, r, w, b, root):
    nb = 1000
    return pl.pallas_call(
        _pre_kernel,
        grid=(_N // nb,),
        in_specs=[
            pl.BlockSpec((nb, _EMB), lambda i: (i, 0)),
            pl.BlockSpec((nb, 1), lambda i: (i, 0)),
            pl.BlockSpec((nb, 1), lambda i: (i, 0)),
            pl.BlockSpec((_EMB, _EMB), lambda i: (0, 0)),
            pl.BlockSpec((1, _EMB), lambda i: (0, 0)),
            pl.BlockSpec((1, _EMB), lambda i: (0, 0)),
        ],
        out_specs=[
            pl.BlockSpec((2, nb, _H), lambda i: (0, i, 0)),
            pl.BlockSpec((nb, _EMB), lambda i: (i, 0)),
        ],
        out_shape=[_f32(2, _N, _H), _f32(_N, _EMB)],
    )(h, dis, r, w, b.reshape(1, _EMB), root)


def _post_kernel(agg_ref, dis_ref, st_ref, hin_ref, z_ref, stats_ref):
    aggf = jnp.concatenate([agg_ref[0], agg_ref[1]], axis=1)
    z = hin_ref[...] + dis_ref[...] * aggf + st_ref[...]
    z_ref[...] = z
    blockstats = jnp.concatenate(
        [jnp.sum(z, axis=0, keepdims=True),
         jnp.sum(z * z, axis=0, keepdims=True)], axis=0)

    @pl.when(pl.program_id(0) == 0)
    def _():
        stats_ref[...] = blockstats

    @pl.when(pl.program_id(0) != 0)
    def _():
        stats_ref[...] = stats_ref[...] + blockstats


def _layer_post(agg, dis, st, hin):
    nb = 1000
    return pl.pallas_call(
        _post_kernel,
        grid=(_N // nb,),
        in_specs=[
            pl.BlockSpec((2, nb, _H), lambda i: (0, i, 0)),
            pl.BlockSpec((nb, 1), lambda i: (i, 0)),
            pl.BlockSpec((nb, _EMB), lambda i: (i, 0)),
            pl.BlockSpec((nb, _EMB), lambda i: (i, 0)),
        ],
        out_specs=[
            pl.BlockSpec((nb, _EMB), lambda i: (i, 0)),
            pl.BlockSpec((2, _EMB), lambda i: (0, 0)),
        ],
        out_shape=[_f32(_N, _EMB), _f32(2, _EMB)],
    )(agg, dis, st, hin)


def _bn_kernel(z_ref, stats_ref, g_ref, b_ref, o_ref, *, relu):
    mu = stats_ref[0:1, :] * (1.0 / _N)
    var = stats_ref[1:2, :] * (1.0 / _N) - mu * mu
    inv = lax.rsqrt(var + 1e-5)
    h = (z_ref[...] - mu) * inv * g_ref[...] + b_ref[...]
    if relu:
        h = jnp.maximum(h, 0.0)
    o_ref[...] = h


def _bn_apply(z, stats, g, b, relu):
    nb = 1000
    return pl.pallas_call(
        functools.partial(_bn_kernel, relu=relu),
        grid=(_N // nb,),
        in_specs=[
            pl.BlockSpec((nb, _EMB), lambda i: (i, 0)),
            pl.BlockSpec((2, _EMB), lambda i: (0, 0)),
            pl.BlockSpec((1, _EMB), lambda i: (0, 0)),
            pl.BlockSpec((1, _EMB), lambda i: (0, 0)),
        ],
        out_specs=pl.BlockSpec((nb, _EMB), lambda i: (i, 0)),
        out_shape=_f32(_N, _EMB),
    )(z, stats, g.reshape(1, _EMB), b.reshape(1, _EMB))


def _pool_kernel(h_ref, b3_ref, sums_ref, cnt_ref):
    ids = b3_ref[0, 0, :]
    io = lax.broadcasted_iota(jnp.int32, (_G, ids.shape[0]), 0)
    oh = (io == ids[None, :]).astype(_F32)
    ps = jnp.dot(oh, h_ref[...], preferred_element_type=_F32)
    pc = jnp.sum(oh, axis=1, keepdims=True)

    @pl.when(pl.program_id(0) == 0)
    def _():
        sums_ref[...] = ps
        cnt_ref[...] = pc

    @pl.when(pl.program_id(0) != 0)
    def _():
        sums_ref[...] = sums_ref[...] + ps
        cnt_ref[...] = cnt_ref[...] + pc


def _pool(h, batch):
    nb = 1000
    batch3 = batch.reshape(_N // nb, 1, nb)
    return pl.pallas_call(
        _pool_kernel,
        grid=(_N // nb,),
        in_specs=[
            pl.BlockSpec((nb, _EMB), lambda i: (i, 0)),
            pl.BlockSpec((1, 1, nb), lambda i: (i, 0, 0)),
        ],
        out_specs=[
            pl.BlockSpec((_G, _EMB), lambda i: (0, 0)),
            pl.BlockSpec((_G, 1), lambda i: (0, 0)),
        ],
        out_shape=[_f32(_G, _EMB), _f32(_G, 1)],
    )(h, batch3)


def _mlp_kernel(s_ref, c_ref, w1_ref, b1_ref, w2_ref, b2_ref, w3_ref, b3_ref,
                o_ref):
    hg = s_ref[...] / jnp.maximum(c_ref[...], 1.0)
    z = jnp.maximum(
        jnp.dot(hg, w1_ref[...].T, preferred_element_type=_F32) + b1_ref[...],
        0.0)
    z = jnp.maximum(
        jnp.dot(z, w2_ref[...].T, preferred_element_type=_F32) + b2_ref[...],
        0.0)
    o_ref[...] = (
        jnp.sum(z * w3_ref[...], axis=1, keepdims=True) + b3_ref[...]
    )


def _mlp(sums, cnt, pred):
    (w1, b1), (w2, b2), (w3, b3) = pred
    hh = w1.shape[0]
    return pl.pallas_call(
        _mlp_kernel,
        out_shape=_f32(_G, 1),
    )(sums, cnt, w1, b1.reshape(1, hh), w2, b2.reshape(1, hh),
      w3, b3.reshape(1, 1))


def _sc_mesh():
    return plsc.VectorSubcoreMesh(core_axis_name="c", subcore_axis_name="s")


def _sc_params():
    return pltpu.CompilerParams(use_tc_tiling_on_sc=False)


def _acc_init(zeros_hbm, acc_sh, s):
    @pl.when(s < _NS - 1)
    def _():
        pltpu.sync_copy(zeros_hbm, acc_sh.at[pl.ds(s * _ZB, _ZB)])

    @pl.when(s == _NS - 1)
    def _():
        pltpu.sync_copy(zeros_hbm.at[pl.ds(0, _ZT)],
                        acc_sh.at[pl.ds((_NS - 1) * _ZB, _ZT)])


def _acc_flush(acc_sh, out_c, s):
    @pl.when(s < _NS - 1)
    def _():
        pltpu.sync_copy(acc_sh.at[pl.ds(s * _ZB, _ZB)],
                        out_c.at[pl.ds(s * _ZB, _ZB)])

    @pl.when(s == _NS - 1)
    def _():
        pltpu.sync_copy(acc_sh.at[pl.ds((_NS - 1) * _ZB, _ZT)],
                        out_c.at[pl.ds((_NS - 1) * _ZB, _ZT)])


def _deg_sc_body(row_hbm, ones_hbm, zeros_hbm, out_hbm, idx_v, ones_v, acc_sh):
    c = lax.axis_index("c")
    s = lax.axis_index("s")
    _acc_init(zeros_hbm, acc_sh, s)
    pltpu.sync_copy(ones_hbm, ones_v)
    plsc.subcore_barrier()
    my_n = 195 + (s < 5).astype(jnp.int32)
    my_base = c * (_NCHUNK // _NC) + s * 195 + jnp.minimum(s, 5)

    @pl.loop(0, 196)
    def _(k):
        @pl.when(k < my_n)
        def _():
            ch = my_base + k
            pltpu.sync_copy(row_hbm.at[pl.ds(ch * _CH, _CH)], idx_v)
            pltpu.sync_copy(ones_v, acc_sh.at[idx_v], add=True)

    plsc.subcore_barrier()
    _acc_flush(acc_sh, out_hbm.at[c], s)


def _deg_sc(row, ones8, zeros8):
    kfn = pl.kernel(
        _deg_sc_body,
        out_type=_f32(_NC, _N, 8),
        mesh=_sc_mesh(),
        compiler_params=_sc_params(),
        scratch_types=[
            pltpu.VMEM((_CH,), jnp.int32),
            pltpu.VMEM((_CH, 8), _F32),
            pltpu.VMEM_SHARED((_N, 8), _F32),
        ],
    )
    return kfn(row, ones8, zeros8)


def _edge_split(s):
    my_n = 390 + (s < 10).astype(jnp.int32)
    my_base = s * 390 + jnp.minimum(s, 10)
    return my_n, my_base


def _prep_sc_body(disw_hbm, ea_hbm, row_hbm, out_hbm, idx_v, drow_v, ea_v):
    c = lax.axis_index("c")
    s = lax.axis_index("s")
    my_n, my_base = _edge_split(s)

    @pl.loop(0, 391)
    def _(k):
        @pl.when(k < my_n)
        def _():
            ch = my_base + k
            pltpu.sync_copy(row_hbm.at[pl.ds(ch * _CH, _CH)], idx_v)
            pltpu.sync_copy(disw_hbm.at[idx_v], drow_v)
            pltpu.sync_copy(ea_hbm.at[c].at[pl.ds(ch * _CH, _CH)], ea_v)

            @pl.loop(0, _CH)
            def _(b):
                for j in range(0, _H, 16):
                    sl = (b, pl.ds(j, 16))
                    ea_v[sl] = drow_v[sl] * ea_v[sl]

            pltpu.sync_copy(ea_v, out_hbm.at[c].at[pl.ds(ch * _CH, _CH)])


def _prep_sc(disw, ea2, row):
    kfn = pl.kernel(
        _prep_sc_body,
        out_type=_f32(_NC, _E, _H),
        mesh=_sc_mesh(),
        compiler_params=_sc_params(),
        scratch_types=[
            pltpu.VMEM((_CH,), jnp.int32),
            pltpu.VMEM((_CH, _H), _F32),
            pltpu.VMEM((_CH, _H), _F32),
        ],
    )
    return kfn(disw, ea2, row)


def _msg_sc_body(g2_hbm, eaw_hbm, row_hbm, col_hbm, zeros_hbm, out_hbm,
                 idxr0, idxr1, idxc0, idxc1, g0, g1, e0, e1,
                 semi0, semi1, semg0, semg1, sems0, sems1, acc_sh):
    c = lax.axis_index("c")
    s = lax.axis_index("s")
    _acc_init(zeros_hbm, acc_sh, s)
    plsc.subcore_barrier()
    my_n, my_base = _edge_split(s)

    idxr = (idxr0, idxr1)
    idxc = (idxc0, idxc1)
    gv = (g0, g1)
    ev = (e0, e1)
    semi = (semi0, semi1)
    semg = (semg0, semg1)
    sems = (sems0, sems1)

    def in_copies(ch, b):
        off = pl.ds(ch * _CH, _CH)
        return (
            pltpu.make_async_copy(row_hbm.at[off], idxr[b], semi[b]),
            pltpu.make_async_copy(col_hbm.at[off], idxc[b], semi[b]),
            pltpu.make_async_copy(eaw_hbm.at[c].at[off], ev[b], semi[b]),
        )

    def issue_inputs(ch, b):
        for cp in in_copies(ch, b):
            cp.start()

    def wait_inputs(ch, b):
        for cp in in_copies(ch, b):
            cp.wait()

    def gather_copy(b):
        return pltpu.make_async_copy(g2_hbm.at[c].at[idxr[b]], gv[b], semg[b])

    def scatter_copy(b):
        return pltpu.make_async_copy(gv[b], acc_sh.at[idxc[b]], sems[b])

    def compute(b):
        @pl.loop(0, _CH)
        def _(rrow):
            for j in range(0, _H, 16):
                sl = (rrow, pl.ds(j, 16))
                gv[b][sl] = jnp.maximum(gv[b][sl] + ev[b][sl], 0.0)

    issue_inputs(my_base, 0)
    wait_inputs(my_base, 0)
    gather_copy(0).start()

    @pl.loop(0, 196)
    def _(kk):
        for b in (0, 1):
            k = kk * 2 + b
            ch = my_base + k

            @pl.when(k < my_n)
            def _():
                gather_copy(b).wait()

                @pl.when(k >= 1)
                def _():
                    scatter_copy(1 - b).wait()

                @pl.when(k + 1 < my_n)
                def _():
                    issue_inputs(ch + 1, 1 - b)

                compute(b)
                scatter_copy(b).start(add=True)

                @pl.when(k + 1 < my_n)
                def _():
                    wait_inputs(ch + 1, 1 - b)
                    gather_copy(1 - b).start()

    @pl.when(my_n == 391)
    def _():
        scatter_copy(0).wait()

    @pl.when(my_n == 390)
    def _():
        scatter_copy(1).wait()

    plsc.subcore_barrier()
    _acc_flush(acc_sh, out_hbm.at[c], s)


def _msg_sc(g2, eaw2, row, col, zeros_h):
    kfn = pl.kernel(
        _msg_sc_body,
        out_type=_f32(_NC, _N, _H),
        mesh=_sc_mesh(),
        compiler_params=_sc_params(),
        scratch_types=[
            pltpu.VMEM((_CH,), jnp.int32),
            pltpu.VMEM((_CH,), jnp.int32),
            pltpu.VMEM((_CH,), jnp.int32),
            pltpu.VMEM((_CH,), jnp.int32),
            pltpu.VMEM((_CH, _H), _F32),
            pltpu.VMEM((_CH, _H), _F32),
            pltpu.VMEM((_CH, _H), _F32),
            pltpu.VMEM((_CH, _H), _F32),
            pltpu.SemaphoreType.DMA,
            pltpu.SemaphoreType.DMA,
            pltpu.SemaphoreType.DMA,
            pltpu.SemaphoreType.DMA,
            pltpu.SemaphoreType.DMA,
            pltpu.SemaphoreType.DMA,
            pltpu.VMEM_SHARED((_N, _H), _F32),
        ],
    )
    return kfn(g2, eaw2, row, col, zeros_h)


def kernel(x, edge_index, edge_attr, batch, params):
    row = edge_index[0]
    col = edge_index[1]

    ones8 = jnp.ones((_CH, 8), _F32)
    zeros8 = jnp.zeros((_ZB, 8), _F32)
    zeros_h = jnp.zeros((_ZB, _H), _F32)

    h = _node_embed(x, params['x_emb_W'], params['x_emb_b'])
    ea2 = _edge_embed(edge_attr, params['edge_emb_W'], params['edge_emb_b'])

    degpart = _deg_sc(row, ones8, zeros8)
    dis, r, disw = _degpost(degpart)

    eaw2 = _prep_sc(disw, ea2, row)

    for l in range(_NL):
        lp = params['layers'][l]
        g2, st = _layer_pre(h, dis, r, lp['lin_W'], lp['lin_b'], lp['root'])
        agg = _msg_sc(g2, eaw2, row, col, zeros_h)
        z, stats = _layer_post(agg, dis, st, h)
        h = _bn_apply(z, stats, lp['bn_g'], lp['bn_b'], relu=(l < _NL - 1))

    sums, cnt = _pool(h, batch)
    return _mlp(sums, cnt, params['pred'])


# speedup vs baseline: 4.8369x; 1.0695x over previous
import functools

import jax
import jax.numpy as jnp
from jax import lax
from jax.experimental import pallas as pl
from jax.experimental.pallas import tpu as pltpu
from jax.experimental.pallas import tpu_sc as plsc

_N = 50000
_E = 800000
_EMB = 64
_G = 128
_NL = 5
_H = 32
_NC = 2
_NS = 16
_CH = 128
_NCHUNK = _E // _CH
_ZB = 3128
_ZT = _N - (_NS - 1) * _ZB

_F32 = jnp.float32


def _f32(*shape):
    return jax.ShapeDtypeStruct(shape, _F32)


def _matmul_bias_kernel(x_ref, w_ref, b_ref, o_ref):
    o_ref[...] = (
        jnp.dot(x_ref[...], w_ref[...].T, preferred_element_type=_F32)
        + b_ref[...]
    )


def _node_embed(x, w, b):
    nb = 1000
    return pl.pallas_call(
        _matmul_bias_kernel,
        grid=(_N // nb,),
        in_specs=[
            pl.BlockSpec((nb, 40), lambda i: (i, 0)),
            pl.BlockSpec((_EMB, 40), lambda i: (0, 0)),
            pl.BlockSpec((1, _EMB), lambda i: (0, 0)),
        ],
        out_specs=pl.BlockSpec((nb, _EMB), lambda i: (i, 0)),
        out_shape=_f32(_N, _EMB),
    )(x, w, b.reshape(1, _EMB))


def _edge_embed_kernel(a_ref, w_ref, b_ref, o_ref):
    ea = (
        jnp.dot(a_ref[...], w_ref[...].T, preferred_element_type=_F32)
        + b_ref[...]
    )
    o_ref[0] = ea[:, :_H]
    o_ref[1] = ea[:, _H:]


def _edge_embed(edge_attr, w, b):
    eb = 2000
    return pl.pallas_call(
        _edge_embed_kernel,
        grid=(_E // eb,),
        in_specs=[
            pl.BlockSpec((eb, 10), lambda i: (i, 0)),
            pl.BlockSpec((_EMB, 10), lambda i: (0, 0)),
            pl.BlockSpec((1, _EMB), lambda i: (0, 0)),
        ],
        out_specs=pl.BlockSpec((2, eb, _H), lambda i: (0, i, 0)),
        out_shape=_f32(2, _E, _H),
    )(edge_attr, w, b.reshape(1, _EMB))


def _degpost_kernel(dp_ref, dis_ref, r_ref, disw_ref):
    d = dp_ref[0, :, 0:1] + dp_ref[1, :, 0:1] + 1.0
    dis = lax.rsqrt(d)
    dis_ref[...] = dis
    r_ref[...] = 1.0 / d
    disw_ref[...] = jnp.broadcast_to(dis, (dis.shape[0], _H))


def _degpost(degpart):
    nb = 1000
    return pl.pallas_call(
        _degpost_kernel,
        grid=(_N // nb,),
        in_specs=[pl.BlockSpec((2, nb, 8), lambda i: (0, i, 0))],
        out_specs=[
            pl.BlockSpec((nb, 1), lambda i: (i, 0)),
            pl.BlockSpec((nb, 1), lambda i: (i, 0)),
            pl.BlockSpec((nb, _H), lambda i: (i, 0)),
        ],
        out_shape=[_f32(_N, 1), _f32(_N, 1), _f32(_N, _H)],
    )(degpart)


def _pre_kernel(h_ref, dis_ref, r_ref, w_ref, b_ref, root_ref, g2_ref, st_ref):
    hp = (
        jnp.dot(h_ref[...], w_ref[...].T, preferred_element_type=_F32)
        + b_ref[...]
    )
    g = hp * dis_ref[...]
    g2_ref[0] = g[:, :_H]
    g2_ref[1] = g[:, _H:]
    st_ref[...] = jnp.maximum(hp + root_ref[...], 0.0) * r_ref[...]


def _layer_pre(h, dis, r, w, b, root):
    nb = 1000
    return pl.pallas_call(
        _pre_kernel,
        grid=(_N // nb,),
        in_specs=[
            pl.BlockSpec((nb, _EMB), lambda i: (i, 0)),
            pl.BlockSpec((nb, 1), lambda i: (i, 0)),
            pl.BlockSpec((nb, 1), lambda i: (i, 0)),
            pl.BlockSpec((_EMB, _EMB), lambda i: (0, 0)),
            pl.BlockSpec((1, _EMB), lambda i: (0, 0)),
            pl.BlockSpec((1, _EMB), lambda i: (0, 0)),
        ],
        out_specs=[
            pl.BlockSpec((2, nb, _H), lambda i: (0, i, 0)),
            pl.BlockSpec((nb, _EMB), lambda i: (i, 0)),
        ],
        out_shape=[_f32(2, _N, _H), _f32(_N, _EMB)],
    )(h, dis, r, w, b.reshape(1, _EMB), root)


def _post_kernel(agg_ref, dis_ref, st_ref, hin_ref, z_ref, stats_ref):
    aggf = jnp.concatenate([agg_ref[0], agg_ref[1]], axis=1)
    z = hin_ref[...] + dis_ref[...] * aggf + st_ref[...]
    z_ref[...] = z
    blockstats = jnp.concatenate(
        [jnp.sum(z, axis=0, keepdims=True),
         jnp.sum(z * z, axis=0, keepdims=True)], axis=0)

    @pl.when(pl.program_id(0) == 0)
    def _():
        stats_ref[...] = blockstats

    @pl.when(pl.program_id(0) != 0)
    def _():
        stats_ref[...] = stats_ref[...] + blockstats


def _layer_post(agg, dis, st, hin):
    nb = 1000
    return pl.pallas_call(
        _post_kernel,
        grid=(_N // nb,),
        in_specs=[
            pl.BlockSpec((2, nb, _H), lambda i: (0, i, 0)),
            pl.BlockSpec((nb, 1), lambda i: (i, 0)),
            pl.BlockSpec((nb, _EMB), lambda i: (i, 0)),
            pl.BlockSpec((nb, _EMB), lambda i: (i, 0)),
        ],
        out_specs=[
            pl.BlockSpec((nb, _EMB), lambda i: (i, 0)),
            pl.BlockSpec((2, _EMB), lambda i: (0, 0)),
        ],
        out_shape=[_f32(_N, _EMB), _f32(2, _EMB)],
    )(agg, dis, st, hin)


def _bn_kernel(z_ref, stats_ref, g_ref, b_ref, o_ref, *, relu):
    mu = stats_ref[0:1, :] * (1.0 / _N)
    var = stats_ref[1:2, :] * (1.0 / _N) - mu * mu
    inv = lax.rsqrt(var + 1e-5)
    h = (z_ref[...] - mu) * inv * g_ref[...] + b_ref[...]
    if relu:
        h = jnp.maximum(h, 0.0)
    o_ref[...] = h


def _bn_apply(z, stats, g, b, relu):
    nb = 1000
    return pl.pallas_call(
        functools.partial(_bn_kernel, relu=relu),
        grid=(_N // nb,),
        in_specs=[
            pl.BlockSpec((nb, _EMB), lambda i: (i, 0)),
            pl.BlockSpec((2, _EMB), lambda i: (0, 0)),
            pl.BlockSpec((1, _EMB), lambda i: (0, 0)),
            pl.BlockSpec((1, _EMB), lambda i: (0, 0)),
        ],
        out_specs=pl.BlockSpec((nb, _EMB), lambda i: (i, 0)),
        out_shape=_f32(_N, _EMB),
    )(z, stats, g.reshape(1, _EMB), b.reshape(1, _EMB))


def _pool_kernel(h_ref, b3_ref, sums_ref, cnt_ref):
    ids = b3_ref[0, 0, :]
    io = lax.broadcasted_iota(jnp.int32, (_G, ids.shape[0]), 0)
    oh = (io == ids[None, :]).astype(_F32)
    ps = jnp.dot(oh, h_ref[...], preferred_element_type=_F32)
    pc = jnp.sum(oh, axis=1, keepdims=True)

    @pl.when(pl.program_id(0) == 0)
    def _():
        sums_ref[...] = ps
        cnt_ref[...] = pc

    @pl.when(pl.program_id(0) != 0)
    def _():
        sums_ref[...] = sums_ref[...] + ps
        cnt_ref[...] = cnt_ref[...] + pc


def _pool(h, batch):
    nb = 1000
    batch3 = batch.reshape(_N // nb, 1, nb)
    return pl.pallas_call(
        _pool_kernel,
        grid=(_N // nb,),
        in_specs=[
            pl.BlockSpec((nb, _EMB), lambda i: (i, 0)),
            pl.BlockSpec((1, 1, nb), lambda i: (i, 0, 0)),
        ],
        out_specs=[
            pl.BlockSpec((_G, _EMB), lambda i: (0, 0)),
            pl.BlockSpec((_G, 1), lambda i: (0, 0)),
        ],
        out_shape=[_f32(_G, _EMB), _f32(_G, 1)],
    )(h, batch3)


def _mlp_kernel(s_ref, c_ref, w1_ref, b1_ref, w2_ref, b2_ref, w3_ref, b3_ref,
                o_ref):
    hg = s_ref[...] / jnp.maximum(c_ref[...], 1.0)
    z = jnp.maximum(
        jnp.dot(hg, w1_ref[...].T, preferred_element_type=_F32) + b1_ref[...],
        0.0)
    z = jnp.maximum(
        jnp.dot(z, w2_ref[...].T, preferred_element_type=_F32) + b2_ref[...],
        0.0)
    o_ref[...] = (
        jnp.sum(z * w3_ref[...], axis=1, keepdims=True) + b3_ref[...]
    )


def _mlp(sums, cnt, pred):
    (w1, b1), (w2, b2), (w3, b3) = pred
    hh = w1.shape[0]
    return pl.pallas_call(
        _mlp_kernel,
        out_shape=_f32(_G, 1),
    )(sums, cnt, w1, b1.reshape(1, hh), w2, b2.reshape(1, hh),
      w3, b3.reshape(1, 1))


def _sc_mesh():
    return plsc.VectorSubcoreMesh(core_axis_name="c", subcore_axis_name="s")


def _sc_params():
    return pltpu.CompilerParams(use_tc_tiling_on_sc=False)


def _acc_init(zeros_hbm, acc_sh, s):
    @pl.when(s < _NS - 1)
    def _():
        pltpu.sync_copy(zeros_hbm, acc_sh.at[pl.ds(s * _ZB, _ZB)])

    @pl.when(s == _NS - 1)
    def _():
        pltpu.sync_copy(zeros_hbm.at[pl.ds(0, _ZT)],
                        acc_sh.at[pl.ds((_NS - 1) * _ZB, _ZT)])


def _acc_flush(acc_sh, out_c, s):
    @pl.when(s < _NS - 1)
    def _():
        pltpu.sync_copy(acc_sh.at[pl.ds(s * _ZB, _ZB)],
                        out_c.at[pl.ds(s * _ZB, _ZB)])

    @pl.when(s == _NS - 1)
    def _():
        pltpu.sync_copy(acc_sh.at[pl.ds((_NS - 1) * _ZB, _ZT)],
                        out_c.at[pl.ds((_NS - 1) * _ZB, _ZT)])


def _deg_sc_body(row_hbm, ones_hbm, zeros_hbm, out_hbm, idx0, idx1, ones_v,
                 semi0, semi1, sems0, sems1, acc_sh):
    c = lax.axis_index("c")
    s = lax.axis_index("s")
    _acc_init(zeros_hbm, acc_sh, s)
    pltpu.sync_copy(ones_hbm, ones_v)
    plsc.subcore_barrier()
    my_n = 195 + (s < 5).astype(jnp.int32)
    my_base = c * (_NCHUNK // _NC) + s * 195 + jnp.minimum(s, 5)

    idx = (idx0, idx1)
    semi = (semi0, semi1)
    sems = (sems0, sems1)

    def idx_copy(ch, b):
        return pltpu.make_async_copy(row_hbm.at[pl.ds(ch * _CH, _CH)],
                                     idx[b], semi[b])

    def scatter_copy(b):
        return pltpu.make_async_copy(ones_v, acc_sh.at[idx[b]], sems[b])

    idx_copy(my_base, 0).start()
    idx_copy(my_base, 0).wait()

    @pl.loop(0, 98)
    def _(kk):
        for b in (0, 1):
            k = kk * 2 + b
            ch = my_base + k

            @pl.when(k < my_n)
            def _():
                @pl.when(k >= 1)
                def _():
                    scatter_copy(1 - b).wait()

                @pl.when(k + 1 < my_n)
                def _():
                    idx_copy(ch + 1, 1 - b).start()

                scatter_copy(b).start(add=True)

                @pl.when(k + 1 < my_n)
                def _():
                    idx_copy(ch + 1, 1 - b).wait()

    @pl.when(my_n == 196)
    def _():
        scatter_copy(1).wait()

    @pl.when(my_n == 195)
    def _():
        scatter_copy(0).wait()

    plsc.subcore_barrier()
    _acc_flush(acc_sh, out_hbm.at[c], s)


def _deg_sc(row, ones8, zeros8):
    kfn = pl.kernel(
        _deg_sc_body,
        out_type=_f32(_NC, _N, 8),
        mesh=_sc_mesh(),
        compiler_params=_sc_params(),
        scratch_types=[
            pltpu.VMEM((_CH,), jnp.int32),
            pltpu.VMEM((_CH,), jnp.int32),
            pltpu.VMEM((_CH, 8), _F32),
            pltpu.SemaphoreType.DMA,
            pltpu.SemaphoreType.DMA,
            pltpu.SemaphoreType.DMA,
            pltpu.SemaphoreType.DMA,
            pltpu.VMEM_SHARED((_N, 8), _F32),
        ],
    )
    return kfn(row, ones8, zeros8)


def _edge_split(s):
    my_n = 390 + (s < 10).astype(jnp.int32)
    my_base = s * 390 + jnp.minimum(s, 10)
    return my_n, my_base


def _prep_sc_body(disw_hbm, ea_hbm, row_hbm, out_hbm,
                  idx0, idx1, d0, d1, e0, e1,
                  semi0, semi1, semg0, semg1, semo0, semo1):
    c = lax.axis_index("c")
    s = lax.axis_index("s")
    my_n, my_base = _edge_split(s)

    idx = (idx0, idx1)
    dv = (d0, d1)
    ev = (e0, e1)
    semi = (semi0, semi1)
    semg = (semg0, semg1)
    semo = (semo0, semo1)

    def in_copies(ch, b):
        off = pl.ds(ch * _CH, _CH)
        return (
            pltpu.make_async_copy(row_hbm.at[off], idx[b], semi[b]),
            pltpu.make_async_copy(ea_hbm.at[c].at[off], ev[b], semi[b]),
        )

    def gather_copy(b):
        return pltpu.make_async_copy(disw_hbm.at[idx[b]], dv[b], semg[b])

    def out_copy(ch, b):
        off = pl.ds(ch * _CH, _CH)
        return pltpu.make_async_copy(ev[b], out_hbm.at[c].at[off], semo[b])

    def compute(b):
        @pl.loop(0, _CH)
        def _(rrow):
            for j in range(0, _H, 16):
                sl = (rrow, pl.ds(j, 16))
                ev[b][sl] = dv[b][sl] * ev[b][sl]

    for cp in in_copies(my_base, 0):
        cp.start()
    for cp in in_copies(my_base, 0):
        cp.wait()
    gather_copy(0).start()

    @pl.loop(0, 196)
    def _(kk):
        for b in (0, 1):
            k = kk * 2 + b
            ch = my_base + k

            @pl.when(k < my_n)
            def _():
                gather_copy(b).wait()

                @pl.when(k >= 1)
                def _():
                    out_copy(ch - 1, 1 - b).wait()

                @pl.when(k + 1 < my_n)
                def _():
                    for cp in in_copies(ch + 1, 1 - b):
                        cp.start()

                compute(b)
                out_copy(ch, b).start()

                @pl.when(k + 1 < my_n)
                def _():
                    for cp in in_copies(ch + 1, 1 - b):
                        cp.wait()
                    gather_copy(1 - b).start()

    @pl.when(my_n == 391)
    def _():
        out_copy(my_base + 390, 0).wait()

    @pl.when(my_n == 390)
    def _():
        out_copy(my_base + 389, 1).wait()


def _prep_sc(disw, ea2, row):
    kfn = pl.kernel(
        _prep_sc_body,
        out_type=_f32(_NC, _E, _H),
        mesh=_sc_mesh(),
        compiler_params=_sc_params(),
        scratch_types=[
            pltpu.VMEM((_CH,), jnp.int32),
            pltpu.VMEM((_CH,), jnp.int32),
            pltpu.VMEM((_CH, _H), _F32),
            pltpu.VMEM((_CH, _H), _F32),
            pltpu.VMEM((_CH, _H), _F32),
            pltpu.VMEM((_CH, _H), _F32),
            pltpu.SemaphoreType.DMA,
            pltpu.SemaphoreType.DMA,
            pltpu.SemaphoreType.DMA,
            pltpu.SemaphoreType.DMA,
            pltpu.SemaphoreType.DMA,
            pltpu.SemaphoreType.DMA,
        ],
    )
    return kfn(disw, ea2, row)


def _msg_sc_body(g2_hbm, eaw_hbm, row_hbm, col_hbm, zeros_hbm, out_hbm,
                 idxr0, idxr1, idxc0, idxc1, g0, g1, e0, e1,
                 semi0, semi1, semg0, semg1, sems0, sems1, acc_sh):
    c = lax.axis_index("c")
    s = lax.axis_index("s")
    _acc_init(zeros_hbm, acc_sh, s)
    plsc.subcore_barrier()
    my_n, my_base = _edge_split(s)

    idxr = (idxr0, idxr1)
    idxc = (idxc0, idxc1)
    gv = (g0, g1)
    ev = (e0, e1)
    semi = (semi0, semi1)
    semg = (semg0, semg1)
    sems = (sems0, sems1)

    def in_copies(ch, b):
        off = pl.ds(ch * _CH, _CH)
        return (
            pltpu.make_async_copy(row_hbm.at[off], idxr[b], semi[b]),
            pltpu.make_async_copy(col_hbm.at[off], idxc[b], semi[b]),
            pltpu.make_async_copy(eaw_hbm.at[c].at[off], ev[b], semi[b]),
        )

    def issue_inputs(ch, b):
        for cp in in_copies(ch, b):
            cp.start()

    def wait_inputs(ch, b):
        for cp in in_copies(ch, b):
            cp.wait()

    def gather_copy(b):
        return pltpu.make_async_copy(g2_hbm.at[c].at[idxr[b]], gv[b], semg[b])

    def scatter_copy(b):
        return pltpu.make_async_copy(gv[b], acc_sh.at[idxc[b]], sems[b])

    def compute(b):
        @pl.loop(0, _CH)
        def _(rrow):
            for j in range(0, _H, 16):
                sl = (rrow, pl.ds(j, 16))
                gv[b][sl] = jnp.maximum(gv[b][sl] + ev[b][sl], 0.0)

    issue_inputs(my_base, 0)
    wait_inputs(my_base, 0)
    gather_copy(0).start()

    @pl.loop(0, 196)
    def _(kk):
        for b in (0, 1):
            k = kk * 2 + b
            ch = my_base + k

            @pl.when(k < my_n)
            def _():
                gather_copy(b).wait()

                @pl.when(k >= 1)
                def _():
                    scatter_copy(1 - b).wait()

                @pl.when(k + 1 < my_n)
                def _():
                    issue_inputs(ch + 1, 1 - b)

                compute(b)
                scatter_copy(b).start(add=True)

                @pl.when(k + 1 < my_n)
                def _():
                    wait_inputs(ch + 1, 1 - b)
                    gather_copy(1 - b).start()

    @pl.when(my_n == 391)
    def _():
        scatter_copy(0).wait()

    @pl.when(my_n == 390)
    def _():
        scatter_copy(1).wait()

    plsc.subcore_barrier()
    _acc_flush(acc_sh, out_hbm.at[c], s)


def _msg_sc(g2, eaw2, row, col, zeros_h):
    kfn = pl.kernel(
        _msg_sc_body,
        out_type=_f32(_NC, _N, _H),
        mesh=_sc_mesh(),
        compiler_params=_sc_params(),
        scratch_types=[
            pltpu.VMEM((_CH,), jnp.int32),
            pltpu.VMEM((_CH,), jnp.int32),
            pltpu.VMEM((_CH,), jnp.int32),
            pltpu.VMEM((_CH,), jnp.int32),
            pltpu.VMEM((_CH, _H), _F32),
            pltpu.VMEM((_CH, _H), _F32),
            pltpu.VMEM((_CH, _H), _F32),
            pltpu.VMEM((_CH, _H), _F32),
            pltpu.SemaphoreType.DMA,
            pltpu.SemaphoreType.DMA,
            pltpu.SemaphoreType.DMA,
            pltpu.SemaphoreType.DMA,
            pltpu.SemaphoreType.DMA,
            pltpu.SemaphoreType.DMA,
            pltpu.VMEM_SHARED((_N, _H), _F32),
        ],
    )
    return kfn(g2, eaw2, row, col, zeros_h)


def kernel(x, edge_index, edge_attr, batch, params):
    row = edge_index[0]
    col = edge_index[1]

    ones8 = jnp.ones((_CH, 8), _F32)
    zeros8 = jnp.zeros((_ZB, 8), _F32)
    zeros_h = jnp.zeros((_ZB, _H), _F32)

    h = _node_embed(x, params['x_emb_W'], params['x_emb_b'])
    ea2 = _edge_embed(edge_attr, params['edge_emb_W'], params['edge_emb_b'])

    degpart = _deg_sc(row, ones8, zeros8)
    dis, r, disw = _degpost(degpart)

    eaw2 = _prep_sc(disw, ea2, row)

    for l in range(_NL):
        lp = params['layers'][l]
        g2, st = _layer_pre(h, dis, r, lp['lin_W'], lp['lin_b'], lp['root'])
        agg = _msg_sc(g2, eaw2, row, col, zeros_h)
        z, stats = _layer_post(agg, dis, st, h)
        h = _bn_apply(z, stats, lp['bn_g'], lp['bn_b'], relu=(l < _NL - 1))

    sums, cnt = _pool(h, batch)
    return _mlp(sums, cnt, params['pred'])


# speedup vs baseline: 5.4808x; 1.1331x over previous
import functools

import jax
import jax.numpy as jnp
from jax import lax
from jax.experimental import pallas as pl
from jax.experimental.pallas import tpu as pltpu
from jax.experimental.pallas import tpu_sc as plsc

_N = 50000
_E = 800000
_EMB = 64
_G = 128
_NL = 5
_H = 32
_NC = 2
_NS = 16
_CH = 128
_CHP = _CH // 4
_NCHUNK = _E // _CH
_ZB = 3128
_ZT = _N - (_NS - 1) * _ZB

_F32 = jnp.float32


def _f32(*shape):
    return jax.ShapeDtypeStruct(shape, _F32)


def _matmul_bias_kernel(x_ref, w_ref, b_ref, o_ref):
    o_ref[...] = (
        jnp.dot(x_ref[...], w_ref[...].T, preferred_element_type=_F32)
        + b_ref[...]
    )


def _node_embed(x, w, b):
    nb = 1000
    return pl.pallas_call(
        _matmul_bias_kernel,
        grid=(_N // nb,),
        in_specs=[
            pl.BlockSpec((nb, 40), lambda i: (i, 0)),
            pl.BlockSpec((_EMB, 40), lambda i: (0, 0)),
            pl.BlockSpec((1, _EMB), lambda i: (0, 0)),
        ],
        out_specs=pl.BlockSpec((nb, _EMB), lambda i: (i, 0)),
        out_shape=_f32(_N, _EMB),
    )(x, w, b.reshape(1, _EMB))


def _edge_embed_kernel(a_ref, w_ref, b_ref, o_ref):
    a4 = a_ref[...]
    for half in range(2):
        wct = w_ref[...][half * _H:(half + 1) * _H, :].T
        cols = []
        for q in range(4):
            z_pre = jnp.zeros((10, q * _H), _F32)
            z_post = jnp.zeros((10, (3 - q) * _H), _F32)
            cols.append(jnp.concatenate(
                [x for x in (z_pre, wct, z_post) if x.shape[1]], axis=1))
        bd = jnp.concatenate(cols, axis=0)
        bias = jnp.concatenate([b_ref[...][:, half * _H:(half + 1) * _H]] * 4,
                               axis=1)
        o_ref[half] = (
            jnp.dot(a4, bd, preferred_element_type=_F32) + bias
        )


def _edge_embed(edge_attr, w, b):
    eb = 4000
    attr4 = edge_attr.reshape(_E // 4, 40)
    return pl.pallas_call(
        _edge_embed_kernel,
        grid=(_E // eb,),
        in_specs=[
            pl.BlockSpec((eb // 4, 40), lambda i: (i, 0)),
            pl.BlockSpec((_EMB, 10), lambda i: (0, 0)),
            pl.BlockSpec((1, _EMB), lambda i: (0, 0)),
        ],
        out_specs=pl.BlockSpec((2, eb // 4, 128), lambda i: (0, i, 0)),
        out_shape=_f32(2, _E // 4, 128),
    )(attr4, w, b.reshape(1, _EMB))


def _degpost_kernel(dp_ref, dis_ref, r_ref, disw_ref):
    d = dp_ref[0, :, 0:1] + dp_ref[1, :, 0:1] + 1.0
    dis = lax.rsqrt(d)
    dis_ref[...] = dis
    r_ref[...] = 1.0 / d
    disw_ref[...] = jnp.broadcast_to(dis, (dis.shape[0], _H))


def _degpost(degpart):
    nb = 1000
    return pl.pallas_call(
        _degpost_kernel,
        grid=(_N // nb,),
        in_specs=[pl.BlockSpec((2, nb, 8), lambda i: (0, i, 0))],
        out_specs=[
            pl.BlockSpec((nb, 1), lambda i: (i, 0)),
            pl.BlockSpec((nb, 1), lambda i: (i, 0)),
            pl.BlockSpec((nb, _H), lambda i: (i, 0)),
        ],
        out_shape=[_f32(_N, 1), _f32(_N, 1), _f32(_N, _H)],
    )(degpart)


def _pre_kernel(h_ref, dis_ref, r_ref, w_ref, b_ref, root_ref, g2_ref, st_ref):
    hp = (
        jnp.dot(h_ref[...], w_ref[...].T, preferred_element_type=_F32)
        + b_ref[...]
    )
    g = hp * dis_ref[...]
    g2_ref[0] = g[:, :_H]
    g2_ref[1] = g[:, _H:]
    st_ref[...] = jnp.maximum(hp + root_ref[...], 0.0) * r_ref[...]


def _layer_pre(h, dis, r, w, b, root):
    nb = 1000
    return pl.pallas_call(
        _pre_kernel,
        grid=(_N // nb,),
        in_specs=[
            pl.BlockSpec((nb, _EMB), lambda i: (i, 0)),
            pl.BlockSpec((nb, 1), lambda i: (i, 0)),
            pl.BlockSpec((nb, 1), lambda i: (i, 0)),
            pl.BlockSpec((_EMB, _EMB), lambda i: (0, 0)),
            pl.BlockSpec((1, _EMB), lambda i: (0, 0)),
            pl.BlockSpec((1, _EMB), lambda i: (0, 0)),
        ],
        out_specs=[
            pl.BlockSpec((2, nb, _H), lambda i: (0, i, 0)),
            pl.BlockSpec((nb, _EMB), lambda i: (i, 0)),
        ],
        out_shape=[_f32(2, _N, _H), _f32(_N, _EMB)],
    )(h, dis, r, w, b.reshape(1, _EMB), root)


def _post_kernel(agg_ref, dis_ref, st_ref, hin_ref, z_ref, stats_ref):
    aggf = jnp.concatenate([agg_ref[0], agg_ref[1]], axis=1)
    z = hin_ref[...] + dis_ref[...] * aggf + st_ref[...]
    z_ref[...] = z
    blockstats = jnp.concatenate(
        [jnp.sum(z, axis=0, keepdims=True),
         jnp.sum(z * z, axis=0, keepdims=True)], axis=0)

    @pl.when(pl.program_id(0) == 0)
    def _():
        stats_ref[...] = blockstats

    @pl.when(pl.program_id(0) != 0)
    def _():
        stats_ref[...] = stats_ref[...] + blockstats


def _layer_post(agg, dis, st, hin):
    nb = 1000
    return pl.pallas_call(
        _post_kernel,
        grid=(_N // nb,),
        in_specs=[
            pl.BlockSpec((2, nb, _H), lambda i: (0, i, 0)),
            pl.BlockSpec((nb, 1), lambda i: (i, 0)),
            pl.BlockSpec((nb, _EMB), lambda i: (i, 0)),
            pl.BlockSpec((nb, _EMB), lambda i: (i, 0)),
        ],
        out_specs=[
            pl.BlockSpec((nb, _EMB), lambda i: (i, 0)),
            pl.BlockSpec((2, _EMB), lambda i: (0, 0)),
        ],
        out_shape=[_f32(_N, _EMB), _f32(2, _EMB)],
    )(agg, dis, st, hin)


def _bn_kernel(z_ref, stats_ref, g_ref, b_ref, o_ref, *, relu):
    mu = stats_ref[0:1, :] * (1.0 / _N)
    var = stats_ref[1:2, :] * (1.0 / _N) - mu * mu
    inv = lax.rsqrt(var + 1e-5)
    h = (z_ref[...] - mu) * inv * g_ref[...] + b_ref[...]
    if relu:
        h = jnp.maximum(h, 0.0)
    o_ref[...] = h


def _bn_apply(z, stats, g, b, relu):
    nb = 1000
    return pl.pallas_call(
        functools.partial(_bn_kernel, relu=relu),
        grid=(_N // nb,),
        in_specs=[
            pl.BlockSpec((nb, _EMB), lambda i: (i, 0)),
            pl.BlockSpec((2, _EMB), lambda i: (0, 0)),
            pl.BlockSpec((1, _EMB), lambda i: (0, 0)),
            pl.BlockSpec((1, _EMB), lambda i: (0, 0)),
        ],
        out_specs=pl.BlockSpec((nb, _EMB), lambda i: (i, 0)),
        out_shape=_f32(_N, _EMB),
    )(z, stats, g.reshape(1, _EMB), b.reshape(1, _EMB))


def _pool_kernel(h_ref, b3_ref, sums_ref, cnt_ref):
    ids = b3_ref[0, 0, :]
    io = lax.broadcasted_iota(jnp.int32, (_G, ids.shape[0]), 0)
    oh = (io == ids[None, :]).astype(_F32)
    ps = jnp.dot(oh, h_ref[...], preferred_element_type=_F32)
    pc = jnp.sum(oh, axis=1, keepdims=True)

    @pl.when(pl.program_id(0) == 0)
    def _():
        sums_ref[...] = ps
        cnt_ref[...] = pc

    @pl.when(pl.program_id(0) != 0)
    def _():
        sums_ref[...] = sums_ref[...] + ps
        cnt_ref[...] = cnt_ref[...] + pc


def _pool(h, batch):
    nb = 1000
    batch3 = batch.reshape(_N // nb, 1, nb)
    return pl.pallas_call(
        _pool_kernel,
        grid=(_N // nb,),
        in_specs=[
            pl.BlockSpec((nb, _EMB), lambda i: (i, 0)),
            pl.BlockSpec((1, 1, nb), lambda i: (i, 0, 0)),
        ],
        out_specs=[
            pl.BlockSpec((_G, _EMB), lambda i: (0, 0)),
            pl.BlockSpec((_G, 1), lambda i: (0, 0)),
        ],
        out_shape=[_f32(_G, _EMB), _f32(_G, 1)],
    )(h, batch3)


def _mlp_kernel(s_ref, c_ref, w1_ref, b1_ref, w2_ref, b2_ref, w3_ref, b3_ref,
                o_ref):
    hg = s_ref[...] / jnp.maximum(c_ref[...], 1.0)
    z = jnp.maximum(
        jnp.dot(hg, w1_ref[...].T, preferred_element_type=_F32) + b1_ref[...],
        0.0)
    z = jnp.maximum(
        jnp.dot(z, w2_ref[...].T, preferred_element_type=_F32) + b2_ref[...],
        0.0)
    o_ref[...] = (
        jnp.sum(z * w3_ref[...], axis=1, keepdims=True) + b3_ref[...]
    )


def _mlp(sums, cnt, pred):
    (w1, b1), (w2, b2), (w3, b3) = pred
    hh = w1.shape[0]
    return pl.pallas_call(
        _mlp_kernel,
        out_shape=_f32(_G, 1),
    )(sums, cnt, w1, b1.reshape(1, hh), w2, b2.reshape(1, hh),
      w3, b3.reshape(1, 1))


def _sc_mesh():
    return plsc.VectorSubcoreMesh(core_axis_name="c", subcore_axis_name="s")


def _sc_params():
    return pltpu.CompilerParams(use_tc_tiling_on_sc=False)


def _acc_init(zeros_hbm, acc_sh, s):
    @pl.when(s < _NS - 1)
    def _():
        pltpu.sync_copy(zeros_hbm, acc_sh.at[pl.ds(s * _ZB, _ZB)])

    @pl.when(s == _NS - 1)
    def _():
        pltpu.sync_copy(zeros_hbm.at[pl.ds(0, _ZT)],
                        acc_sh.at[pl.ds((_NS - 1) * _ZB, _ZT)])


def _acc_flush(acc_sh, out_c, s):
    @pl.when(s < _NS - 1)
    def _():
        pltpu.sync_copy(acc_sh.at[pl.ds(s * _ZB, _ZB)],
                        out_c.at[pl.ds(s * _ZB, _ZB)])

    @pl.when(s == _NS - 1)
    def _():
        pltpu.sync_copy(acc_sh.at[pl.ds((_NS - 1) * _ZB, _ZT)],
                        out_c.at[pl.ds((_NS - 1) * _ZB, _ZT)])


def _deg_sc_body(row_hbm, ones_hbm, zeros_hbm, out_hbm, idx0, idx1, ones_v,
                 semi0, semi1, sems0, sems1, acc_sh):
    c = lax.axis_index("c")
    s = lax.axis_index("s")
    _acc_init(zeros_hbm, acc_sh, s)
    pltpu.sync_copy(ones_hbm, ones_v)
    plsc.subcore_barrier()
    my_n = 195 + (s < 5).astype(jnp.int32)
    my_base = c * (_NCHUNK // _NC) + s * 195 + jnp.minimum(s, 5)

    idx = (idx0, idx1)
    semi = (semi0, semi1)
    sems = (sems0, sems1)

    def idx_copy(ch, b):
        return pltpu.make_async_copy(row_hbm.at[pl.ds(ch * _CH, _CH)],
                                     idx[b], semi[b])

    def scatter_copy(b):
        return pltpu.make_async_copy(ones_v, acc_sh.at[idx[b]], sems[b])

    idx_copy(my_base, 0).start()
    idx_copy(my_base, 0).wait()

    @pl.loop(0, 98)
    def _(kk):
        for b in (0, 1):
            k = kk * 2 + b
            ch = my_base + k

            @pl.when(k < my_n)
            def _():
                @pl.when(k >= 1)
                def _():
                    scatter_copy(1 - b).wait()

                @pl.when(k + 1 < my_n)
                def _():
                    idx_copy(ch + 1, 1 - b).start()

                scatter_copy(b).start(add=True)

                @pl.when(k + 1 < my_n)
                def _():
                    idx_copy(ch + 1, 1 - b).wait()

    @pl.when(my_n == 196)
    def _():
        scatter_copy(1).wait()

    @pl.when(my_n == 195)
    def _():
        scatter_copy(0).wait()

    plsc.subcore_barrier()
    _acc_flush(acc_sh, out_hbm.at[c], s)


def _deg_sc(row, ones8, zeros8):
    kfn = pl.kernel(
        _deg_sc_body,
        out_type=_f32(_NC, _N, 8),
        mesh=_sc_mesh(),
        compiler_params=_sc_params(),
        scratch_types=[
            pltpu.VMEM((_CH,), jnp.int32),
            pltpu.VMEM((_CH,), jnp.int32),
            pltpu.VMEM((_CH, 8), _F32),
            pltpu.SemaphoreType.DMA,
            pltpu.SemaphoreType.DMA,
            pltpu.SemaphoreType.DMA,
            pltpu.SemaphoreType.DMA,
            pltpu.VMEM_SHARED((_N, 8), _F32),
        ],
    )
    return kfn(row, ones8, zeros8)


def _edge_split(s):
    my_n = 390 + (s < 10).astype(jnp.int32)
    my_base = s * 390 + jnp.minimum(s, 10)
    return my_n, my_base


def _prep_sc_body(disw_hbm, ea_hbm, row_hbm, out_hbm,
                  idx0, idx1, d0, d1, e0, e1,
                  semi0, semi1, semg0, semg1, semo0, semo1):
    c = lax.axis_index("c")
    s = lax.axis_index("s")
    my_n, my_base = _edge_split(s)

    idx = (idx0, idx1)
    dv = (d0, d1)
    ev = (e0, e1)
    semi = (semi0, semi1)
    semg = (semg0, semg1)
    semo = (semo0, semo1)

    def in_copies(ch, b):
        return (
            pltpu.make_async_copy(row_hbm.at[pl.ds(ch * _CH, _CH)],
                                  idx[b], semi[b]),
            pltpu.make_async_copy(ea_hbm.at[c].at[pl.ds(ch * _CHP, _CHP)],
                                  ev[b], semi[b]),
        )

    def gather_copy(b):
        return pltpu.make_async_copy(disw_hbm.at[idx[b]], dv[b], semg[b])

    def out_copy(ch, b):
        off = pl.ds(ch * _CHP, _CHP)
        return pltpu.make_async_copy(ev[b], out_hbm.at[c].at[off], semo[b])

    def compute(b):
        @pl.loop(0, _CHP)
        def _(rrow):
            for q in range(4):
                for j in range(0, _H, 16):
                    sle = (rrow, pl.ds(q * _H + j, 16))
                    sld = (rrow * 4 + q, pl.ds(j, 16))
                    ev[b][sle] = dv[b][sld] * ev[b][sle]

    for cp in in_copies(my_base, 0):
        cp.start()
    for cp in in_copies(my_base, 0):
        cp.wait()
    gather_copy(0).start()

    @pl.loop(0, 196)
    def _(kk):
        for b in (0, 1):
            k = kk * 2 + b
            ch = my_base + k

            @pl.when(k < my_n)
            def _():
                gather_copy(b).wait()

                @pl.when(k >= 1)
                def _():
                    out_copy(ch - 1, 1 - b).wait()

                @pl.when(k + 1 < my_n)
                def _():
                    for cp in in_copies(ch + 1, 1 - b):
                        cp.start()

                compute(b)
                out_copy(ch, b).start()

                @pl.when(k + 1 < my_n)
                def _():
                    for cp in in_copies(ch + 1, 1 - b):
                        cp.wait()
                    gather_copy(1 - b).start()

    @pl.when(my_n == 391)
    def _():
        out_copy(my_base + 390, 0).wait()

    @pl.when(my_n == 390)
    def _():
        out_copy(my_base + 389, 1).wait()


def _prep_sc(disw, ea2, row):
    kfn = pl.kernel(
        _prep_sc_body,
        out_type=_f32(_NC, _E // 4, 128),
        mesh=_sc_mesh(),
        compiler_params=_sc_params(),
        scratch_types=[
            pltpu.VMEM((_CH,), jnp.int32),
            pltpu.VMEM((_CH,), jnp.int32),
            pltpu.VMEM((_CH, _H), _F32),
            pltpu.VMEM((_CH, _H), _F32),
            pltpu.VMEM((_CHP, 128), _F32),
            pltpu.VMEM((_CHP, 128), _F32),
            pltpu.SemaphoreType.DMA,
            pltpu.SemaphoreType.DMA,
            pltpu.SemaphoreType.DMA,
            pltpu.SemaphoreType.DMA,
            pltpu.SemaphoreType.DMA,
            pltpu.SemaphoreType.DMA,
        ],
    )
    return kfn(disw, ea2, row)


def _msg_sc_body(g2_hbm, eaw_hbm, row_hbm, col_hbm, zeros_hbm, out_hbm,
                 idxr0, idxr1, idxc0, idxc1, g0, g1, e0, e1,
                 semi0, semi1, semg0, semg1, sems0, sems1, acc_sh):
    c = lax.axis_index("c")
    s = lax.axis_index("s")
    _acc_init(zeros_hbm, acc_sh, s)
    plsc.subcore_barrier()
    my_n, my_base = _edge_split(s)

    idxr = (idxr0, idxr1)
    idxc = (idxc0, idxc1)
    gv = (g0, g1)
    ev = (e0, e1)
    semi = (semi0, semi1)
    semg = (semg0, semg1)
    sems = (sems0, sems1)

    def in_copies(ch, b):
        off = pl.ds(ch * _CH, _CH)
        return (
            pltpu.make_async_copy(row_hbm.at[off], idxr[b], semi[b]),
            pltpu.make_async_copy(col_hbm.at[off], idxc[b], semi[b]),
            pltpu.make_async_copy(eaw_hbm.at[c].at[pl.ds(ch * _CHP, _CHP)],
                                  ev[b], semi[b]),
        )

    def issue_inputs(ch, b):
        for cp in in_copies(ch, b):
            cp.start()

    def wait_inputs(ch, b):
        for cp in in_copies(ch, b):
            cp.wait()

    def gather_copy(b):
        return pltpu.make_async_copy(g2_hbm.at[c].at[idxr[b]], gv[b], semg[b])

    def scatter_copy(b):
        return pltpu.make_async_copy(gv[b], acc_sh.at[idxc[b]], sems[b])

    def compute(b):
        @pl.loop(0, _CHP)
        def _(rrow):
            for q in range(4):
                for j in range(0, _H, 16):
                    slg = (rrow * 4 + q, pl.ds(j, 16))
                    sle = (rrow, pl.ds(q * _H + j, 16))
                    gv[b][slg] = jnp.maximum(gv[b][slg] + ev[b][sle], 0.0)

    issue_inputs(my_base, 0)
    wait_inputs(my_base, 0)
    gather_copy(0).start()

    @pl.loop(0, 196)
    def _(kk):
        for b in (0, 1):
            k = kk * 2 + b
            ch = my_base + k

            @pl.when(k < my_n)
            def _():
                gather_copy(b).wait()

                @pl.when(k >= 1)
                def _():
                    scatter_copy(1 - b).wait()

                @pl.when(k + 1 < my_n)
                def _():
                    issue_inputs(ch + 1, 1 - b)

                compute(b)
                scatter_copy(b).start(add=True)

                @pl.when(k + 1 < my_n)
                def _():
                    wait_inputs(ch + 1, 1 - b)
                    gather_copy(1 - b).start()

    @pl.when(my_n == 391)
    def _():
        scatter_copy(0).wait()

    @pl.when(my_n == 390)
    def _():
        scatter_copy(1).wait()

    plsc.subcore_barrier()
    _acc_flush(acc_sh, out_hbm.at[c], s)


def _msg_sc(g2, eaw2, row, col, zeros_h):
    kfn = pl.kernel(
        _msg_sc_body,
        out_type=_f32(_NC, _N, _H),
        mesh=_sc_mesh(),
        compiler_params=_sc_params(),
        scratch_types=[
            pltpu.VMEM((_CH,), jnp.int32),
            pltpu.VMEM((_CH,), jnp.int32),
            pltpu.VMEM((_CH,), jnp.int32),
            pltpu.VMEM((_CH,), jnp.int32),
            pltpu.VMEM((_CH, _H), _F32),
            pltpu.VMEM((_CH, _H), _F32),
            pltpu.VMEM((_CHP, 128), _F32),
            pltpu.VMEM((_CHP, 128), _F32),
            pltpu.SemaphoreType.DMA,
            pltpu.SemaphoreType.DMA,
            pltpu.SemaphoreType.DMA,
            pltpu.SemaphoreType.DMA,
            pltpu.SemaphoreType.DMA,
            pltpu.SemaphoreType.DMA,
            pltpu.VMEM_SHARED((_N, _H), _F32),
        ],
    )
    return kfn(g2, eaw2, row, col, zeros_h)


def kernel(x, edge_index, edge_attr, batch, params):
    row = edge_index[0]
    col = edge_index[1]

    ones8 = jnp.ones((_CH, 8), _F32)
    zeros8 = jnp.zeros((_ZB, 8), _F32)
    zeros_h = jnp.zeros((_ZB, _H), _F32)

    h = _node_embed(x, params['x_emb_W'], params['x_emb_b'])
    ea2 = _edge_embed(edge_attr, params['edge_emb_W'], params['edge_emb_b'])

    degpart = _deg_sc(row, ones8, zeros8)
    dis, r, disw = _degpost(degpart)

    eaw2 = _prep_sc(disw, ea2, row)

    for l in range(_NL):
        lp = params['layers'][l]
        g2, st = _layer_pre(h, dis, r, lp['lin_W'], lp['lin_b'], lp['root'])
        agg = _msg_sc(g2, eaw2, row, col, zeros_h)
        z, stats = _layer_post(agg, dis, st, h)
        h = _bn_apply(z, stats, lp['bn_g'], lp['bn_b'], relu=(l < _NL - 1))

    sums, cnt = _pool(h, batch)
    return _mlp(sums, cnt, params['pred'])


# speedup vs baseline: 5.4852x; 1.0008x over previous
import functools

import jax
import jax.numpy as jnp
from jax import lax
from jax.experimental import pallas as pl
from jax.experimental.pallas import tpu as pltpu
from jax.experimental.pallas import tpu_sc as plsc

_N = 50000
_E = 800000
_EMB = 64
_G = 128
_NL = 5
_H = 32
_NC = 2
_NS = 16
_CH = 128
_CHP = _CH // 4
_NCHUNK = _E // _CH
_ZB = 3128
_ZT = _N - (_NS - 1) * _ZB

_F32 = jnp.float32


def _f32(*shape):
    return jax.ShapeDtypeStruct(shape, _F32)


def _matmul_bias_kernel(x_ref, w_ref, b_ref, o_ref):
    o_ref[...] = (
        jnp.dot(x_ref[...], w_ref[...].T, preferred_element_type=_F32)
        + b_ref[...]
    )


def _node_embed(x, w, b):
    nb = 1000
    return pl.pallas_call(
        _matmul_bias_kernel,
        grid=(_N // nb,),
        in_specs=[
            pl.BlockSpec((nb, 40), lambda i: (i, 0)),
            pl.BlockSpec((_EMB, 40), lambda i: (0, 0)),
            pl.BlockSpec((1, _EMB), lambda i: (0, 0)),
        ],
        out_specs=pl.BlockSpec((nb, _EMB), lambda i: (i, 0)),
        out_shape=_f32(_N, _EMB),
    )(x, w, b.reshape(1, _EMB))


def _edge_embed_kernel(a_ref, w_ref, b_ref, o_ref):
    a4 = a_ref[...]
    for half in range(2):
        wct = w_ref[...][half * _H:(half + 1) * _H, :].T
        cols = []
        for q in range(4):
            z_pre = jnp.zeros((10, q * _H), _F32)
            z_post = jnp.zeros((10, (3 - q) * _H), _F32)
            cols.append(jnp.concatenate(
                [x for x in (z_pre, wct, z_post) if x.shape[1]], axis=1))
        bd = jnp.concatenate(cols, axis=0)
        bias = jnp.concatenate([b_ref[...][:, half * _H:(half + 1) * _H]] * 4,
                               axis=1)
        o_ref[half] = (
            jnp.dot(a4, bd, preferred_element_type=_F32) + bias
        )


def _edge_embed(edge_attr, w, b):
    eb = 4000
    attr4 = edge_attr.reshape(_E // 4, 40)
    return pl.pallas_call(
        _edge_embed_kernel,
        grid=(_E // eb,),
        in_specs=[
            pl.BlockSpec((eb // 4, 40), lambda i: (i, 0)),
            pl.BlockSpec((_EMB, 10), lambda i: (0, 0)),
            pl.BlockSpec((1, _EMB), lambda i: (0, 0)),
        ],
        out_specs=pl.BlockSpec((2, eb // 4, 128), lambda i: (0, i, 0)),
        out_shape=_f32(2, _E // 4, 128),
    )(attr4, w, b.reshape(1, _EMB))


def _degpost_kernel(dp_ref, dis_ref, r_ref, disw_ref):
    d = dp_ref[0, :, 0:1] + dp_ref[1, :, 0:1] + 1.0
    dis = lax.rsqrt(d)
    dis_ref[...] = dis
    r_ref[...] = 1.0 / d
    disw_ref[...] = jnp.broadcast_to(dis, (dis.shape[0], _H))


def _degpost(degpart):
    nb = 1000
    return pl.pallas_call(
        _degpost_kernel,
        grid=(_N // nb,),
        in_specs=[pl.BlockSpec((2, nb, 8), lambda i: (0, i, 0))],
        out_specs=[
            pl.BlockSpec((nb, 1), lambda i: (i, 0)),
            pl.BlockSpec((nb, 1), lambda i: (i, 0)),
            pl.BlockSpec((nb, _H), lambda i: (i, 0)),
        ],
        out_shape=[_f32(_N, 1), _f32(_N, 1), _f32(_N, _H)],
    )(degpart)


def _pre_kernel(h_ref, dis_ref, r_ref, w_ref, b_ref, root_ref, g2_ref, st_ref):
    hp = (
        jnp.dot(h_ref[...], w_ref[...].T, preferred_element_type=_F32)
        + b_ref[...]
    )
    g = hp * dis_ref[...]
    g2_ref[0] = g[:, :_H]
    g2_ref[1] = g[:, _H:]
    st_ref[...] = jnp.maximum(hp + root_ref[...], 0.0) * r_ref[...]


def _layer_pre(h, dis, r, w, b, root):
    nb = 1000
    return pl.pallas_call(
        _pre_kernel,
        grid=(_N // nb,),
        in_specs=[
            pl.BlockSpec((nb, _EMB), lambda i: (i, 0)),
            pl.BlockSpec((nb, 1), lambda i: (i, 0)),
            pl.BlockSpec((nb, 1), lambda i: (i, 0)),
            pl.BlockSpec((_EMB, _EMB), lambda i: (0, 0)),
            pl.BlockSpec((1, _EMB), lambda i: (0, 0)),
            pl.BlockSpec((1, _EMB), lambda i: (0, 0)),
        ],
        out_specs=[
            pl.BlockSpec((2, nb, _H), lambda i: (0, i, 0)),
            pl.BlockSpec((nb, _EMB), lambda i: (i, 0)),
        ],
        out_shape=[_f32(2, _N, _H), _f32(_N, _EMB)],
    )(h, dis, r, w, b.reshape(1, _EMB), root)


def _post_kernel(agg_ref, dis_ref, st_ref, hin_ref, z_ref, stats_ref):
    aggf = jnp.concatenate([agg_ref[0], agg_ref[1]], axis=1)
    z = hin_ref[...] + dis_ref[...] * aggf + st_ref[...]
    z_ref[...] = z
    blockstats = jnp.concatenate(
        [jnp.sum(z, axis=0, keepdims=True),
         jnp.sum(z * z, axis=0, keepdims=True)], axis=0)

    @pl.when(pl.program_id(0) == 0)
    def _():
        stats_ref[...] = blockstats

    @pl.when(pl.program_id(0) != 0)
    def _():
        stats_ref[...] = stats_ref[...] + blockstats


def _layer_post(agg, dis, st, hin):
    nb = 1000
    return pl.pallas_call(
        _post_kernel,
        grid=(_N // nb,),
        in_specs=[
            pl.BlockSpec((2, nb, _H), lambda i: (0, i, 0)),
            pl.BlockSpec((nb, 1), lambda i: (i, 0)),
            pl.BlockSpec((nb, _EMB), lambda i: (i, 0)),
            pl.BlockSpec((nb, _EMB), lambda i: (i, 0)),
        ],
        out_specs=[
            pl.BlockSpec((nb, _EMB), lambda i: (i, 0)),
            pl.BlockSpec((2, _EMB), lambda i: (0, 0)),
        ],
        out_shape=[_f32(_N, _EMB), _f32(2, _EMB)],
    )(agg, dis, st, hin)


def _bn_kernel(z_ref, stats_ref, g_ref, b_ref, o_ref, *, relu):
    mu = stats_ref[0:1, :] * (1.0 / _N)
    var = stats_ref[1:2, :] * (1.0 / _N) - mu * mu
    inv = lax.rsqrt(var + 1e-5)
    h = (z_ref[...] - mu) * inv * g_ref[...] + b_ref[...]
    if relu:
        h = jnp.maximum(h, 0.0)
    o_ref[...] = h


def _bn_apply(z, stats, g, b, relu):
    nb = 1000
    return pl.pallas_call(
        functools.partial(_bn_kernel, relu=relu),
        grid=(_N // nb,),
        in_specs=[
            pl.BlockSpec((nb, _EMB), lambda i: (i, 0)),
            pl.BlockSpec((2, _EMB), lambda i: (0, 0)),
            pl.BlockSpec((1, _EMB), lambda i: (0, 0)),
            pl.BlockSpec((1, _EMB), lambda i: (0, 0)),
        ],
        out_specs=pl.BlockSpec((nb, _EMB), lambda i: (i, 0)),
        out_shape=_f32(_N, _EMB),
    )(z, stats, g.reshape(1, _EMB), b.reshape(1, _EMB))


def _pool_kernel(h_ref, b3_ref, sums_ref, cnt_ref):
    ids = b3_ref[0, 0, :]
    io = lax.broadcasted_iota(jnp.int32, (_G, ids.shape[0]), 0)
    oh = (io == ids[None, :]).astype(_F32)
    ps = jnp.dot(oh, h_ref[...], preferred_element_type=_F32)
    pc = jnp.sum(oh, axis=1, keepdims=True)

    @pl.when(pl.program_id(0) == 0)
    def _():
        sums_ref[...] = ps
        cnt_ref[...] = pc

    @pl.when(pl.program_id(0) != 0)
    def _():
        sums_ref[...] = sums_ref[...] + ps
        cnt_ref[...] = cnt_ref[...] + pc


def _pool(h, batch):
    nb = 1000
    batch3 = batch.reshape(_N // nb, 1, nb)
    return pl.pallas_call(
        _pool_kernel,
        grid=(_N // nb,),
        in_specs=[
            pl.BlockSpec((nb, _EMB), lambda i: (i, 0)),
            pl.BlockSpec((1, 1, nb), lambda i: (i, 0, 0)),
        ],
        out_specs=[
            pl.BlockSpec((_G, _EMB), lambda i: (0, 0)),
            pl.BlockSpec((_G, 1), lambda i: (0, 0)),
        ],
        out_shape=[_f32(_G, _EMB), _f32(_G, 1)],
    )(h, batch3)


def _mlp_kernel(s_ref, c_ref, w1_ref, b1_ref, w2_ref, b2_ref, w3_ref, b3_ref,
                o_ref):
    hg = s_ref[...] / jnp.maximum(c_ref[...], 1.0)
    z = jnp.maximum(
        jnp.dot(hg, w1_ref[...].T, preferred_element_type=_F32) + b1_ref[...],
        0.0)
    z = jnp.maximum(
        jnp.dot(z, w2_ref[...].T, preferred_element_type=_F32) + b2_ref[...],
        0.0)
    o_ref[...] = (
        jnp.sum(z * w3_ref[...], axis=1, keepdims=True) + b3_ref[...]
    )


def _mlp(sums, cnt, pred):
    (w1, b1), (w2, b2), (w3, b3) = pred
    hh = w1.shape[0]
    return pl.pallas_call(
        _mlp_kernel,
        out_shape=_f32(_G, 1),
    )(sums, cnt, w1, b1.reshape(1, hh), w2, b2.reshape(1, hh),
      w3, b3.reshape(1, 1))


def _sc_mesh():
    return plsc.VectorSubcoreMesh(core_axis_name="c", subcore_axis_name="s")


def _sc_params():
    return pltpu.CompilerParams(use_tc_tiling_on_sc=False)


def _acc_init(zeros_hbm, acc_sh, s):
    @pl.when(s < _NS - 1)
    def _():
        pltpu.sync_copy(zeros_hbm, acc_sh.at[pl.ds(s * _ZB, _ZB)])

    @pl.when(s == _NS - 1)
    def _():
        pltpu.sync_copy(zeros_hbm.at[pl.ds(0, _ZT)],
                        acc_sh.at[pl.ds((_NS - 1) * _ZB, _ZT)])


def _acc_flush(acc_sh, out_c, s):
    @pl.when(s < _NS - 1)
    def _():
        pltpu.sync_copy(acc_sh.at[pl.ds(s * _ZB, _ZB)],
                        out_c.at[pl.ds(s * _ZB, _ZB)])

    @pl.when(s == _NS - 1)
    def _():
        pltpu.sync_copy(acc_sh.at[pl.ds((_NS - 1) * _ZB, _ZT)],
                        out_c.at[pl.ds((_NS - 1) * _ZB, _ZT)])


def _deg_sc_body(row_hbm, ones_hbm, zeros_hbm, out_hbm, idx0, idx1, ones_v,
                 semi0, semi1, sems0, sems1, acc_sh):
    c = lax.axis_index("c")
    s = lax.axis_index("s")
    _acc_init(zeros_hbm, acc_sh, s)
    pltpu.sync_copy(ones_hbm, ones_v)
    plsc.subcore_barrier()
    my_n = 195 + (s < 5).astype(jnp.int32)
    my_base = c * (_NCHUNK // _NC) + s * 195 + jnp.minimum(s, 5)

    idx = (idx0, idx1)
    semi = (semi0, semi1)
    sems = (sems0, sems1)

    def idx_copy(ch, b):
        return pltpu.make_async_copy(row_hbm.at[pl.ds(ch * _CH, _CH)],
                                     idx[b], semi[b])

    def scatter_copy(b):
        return pltpu.make_async_copy(ones_v, acc_sh.at[idx[b]], sems[b])

    idx_copy(my_base, 0).start()
    idx_copy(my_base, 0).wait()

    @pl.loop(0, 98)
    def _(kk):
        for b in (0, 1):
            k = kk * 2 + b
            ch = my_base + k

            @pl.when(k < my_n)
            def _():
                @pl.when(k >= 1)
                def _():
                    scatter_copy(1 - b).wait()

                @pl.when(k + 1 < my_n)
                def _():
                    idx_copy(ch + 1, 1 - b).start()

                scatter_copy(b).start(add=True)

                @pl.when(k + 1 < my_n)
                def _():
                    idx_copy(ch + 1, 1 - b).wait()

    @pl.when(my_n == 196)
    def _():
        scatter_copy(1).wait()

    @pl.when(my_n == 195)
    def _():
        scatter_copy(0).wait()

    plsc.subcore_barrier()
    _acc_flush(acc_sh, out_hbm.at[c], s)


def _deg_sc(row, ones8, zeros8):
    kfn = pl.kernel(
        _deg_sc_body,
        out_type=_f32(_NC, _N, 8),
        mesh=_sc_mesh(),
        compiler_params=_sc_params(),
        scratch_types=[
            pltpu.VMEM((_CH,), jnp.int32),
            pltpu.VMEM((_CH,), jnp.int32),
            pltpu.VMEM((_CH, 8), _F32),
            pltpu.SemaphoreType.DMA,
            pltpu.SemaphoreType.DMA,
            pltpu.SemaphoreType.DMA,
            pltpu.SemaphoreType.DMA,
            pltpu.VMEM_SHARED((_N, 8), _F32),
        ],
    )
    return kfn(row, ones8, zeros8)


def _edge_split(s):
    my_n = 390 + (s < 10).astype(jnp.int32)
    my_base = s * 390 + jnp.minimum(s, 10)
    return my_n, my_base


def _prep_sc_body(disw_hbm, ea_hbm, row_hbm, out_hbm,
                  idx0, idx1, d0, d1, e0, e1,
                  semi0, semi1, semg0, semg1, semo0, semo1):
    c = lax.axis_index("c")
    s = lax.axis_index("s")
    my_n, my_base = _edge_split(s)

    idx = (idx0, idx1)
    dv = (d0, d1)
    ev = (e0, e1)
    semi = (semi0, semi1)
    semg = (semg0, semg1)
    semo = (semo0, semo1)

    def in_copies(ch, b):
        return (
            pltpu.make_async_copy(row_hbm.at[pl.ds(ch * _CH, _CH)],
                                  idx[b], semi[b]),
            pltpu.make_async_copy(ea_hbm.at[c].at[pl.ds(ch * _CHP, _CHP)],
                                  ev[b], semi[b]),
        )

    def gather_copy(b):
        return pltpu.make_async_copy(disw_hbm.at[idx[b]], dv[b], semg[b])

    def out_copy(ch, b):
        off = pl.ds(ch * _CHP, _CHP)
        return pltpu.make_async_copy(ev[b], out_hbm.at[c].at[off], semo[b])

    def compute(b):
        @plsc.parallel_loop(0, _CHP, unroll=4)
        def _(rrow):
            for q in range(4):
                for j in range(0, _H, 16):
                    sle = (rrow, pl.ds(q * _H + j, 16))
                    sld = (rrow * 4 + q, pl.ds(j, 16))
                    ev[b][sle] = dv[b][sld] * ev[b][sle]

    for cp in in_copies(my_base, 0):
        cp.start()
    for cp in in_copies(my_base, 0):
        cp.wait()
    gather_copy(0).start()

    @pl.loop(0, 196)
    def _(kk):
        for b in (0, 1):
            k = kk * 2 + b
            ch = my_base + k

            @pl.when(k < my_n)
            def _():
                gather_copy(b).wait()

                @pl.when(k >= 1)
                def _():
                    out_copy(ch - 1, 1 - b).wait()

                @pl.when(k + 1 < my_n)
                def _():
                    for cp in in_copies(ch + 1, 1 - b):
                        cp.start()

                compute(b)
                out_copy(ch, b).start()

                @pl.when(k + 1 < my_n)
                def _():
                    for cp in in_copies(ch + 1, 1 - b):
                        cp.wait()
                    gather_copy(1 - b).start()

    @pl.when(my_n == 391)
    def _():
        out_copy(my_base + 390, 0).wait()

    @pl.when(my_n == 390)
    def _():
        out_copy(my_base + 389, 1).wait()


def _prep_sc(disw, ea2, row):
    kfn = pl.kernel(
        _prep_sc_body,
        out_type=_f32(_NC, _E // 4, 128),
        mesh=_sc_mesh(),
        compiler_params=_sc_params(),
        scratch_types=[
            pltpu.VMEM((_CH,), jnp.int32),
            pltpu.VMEM((_CH,), jnp.int32),
            pltpu.VMEM((_CH, _H), _F32),
            pltpu.VMEM((_CH, _H), _F32),
            pltpu.VMEM((_CHP, 128), _F32),
            pltpu.VMEM((_CHP, 128), _F32),
            pltpu.SemaphoreType.DMA,
            pltpu.SemaphoreType.DMA,
            pltpu.SemaphoreType.DMA,
            pltpu.SemaphoreType.DMA,
            pltpu.SemaphoreType.DMA,
            pltpu.SemaphoreType.DMA,
        ],
    )
    return kfn(disw, ea2, row)


def _msg_sc_body(g2_hbm, eaw_hbm, row_hbm, col_hbm, zeros_hbm, out_hbm,
                 idxr0, idxr1, idxc0, idxc1, g0, g1, e0, e1,
                 semi0, semi1, semg0, semg1, sems0, sems1, acc_sh):
    c = lax.axis_index("c")
    s = lax.axis_index("s")
    _acc_init(zeros_hbm, acc_sh, s)
    plsc.subcore_barrier()
    my_n, my_base = _edge_split(s)

    idxr = (idxr0, idxr1)
    idxc = (idxc0, idxc1)
    gv = (g0, g1)
    ev = (e0, e1)
    semi = (semi0, semi1)
    semg = (semg0, semg1)
    sems = (sems0, sems1)

    def in_copies(ch, b):
        off = pl.ds(ch * _CH, _CH)
        return (
            pltpu.make_async_copy(row_hbm.at[off], idxr[b], semi[b]),
            pltpu.make_async_copy(col_hbm.at[off], idxc[b], semi[b]),
            pltpu.make_async_copy(eaw_hbm.at[c].at[pl.ds(ch * _CHP, _CHP)],
                                  ev[b], semi[b]),
        )

    def issue_inputs(ch, b):
        for cp in in_copies(ch, b):
            cp.start()

    def wait_inputs(ch, b):
        for cp in in_copies(ch, b):
            cp.wait()

    def gather_copy(b):
        return pltpu.make_async_copy(g2_hbm.at[c].at[idxr[b]], gv[b], semg[b])

    def scatter_copy(b):
        return pltpu.make_async_copy(gv[b], acc_sh.at[idxc[b]], sems[b])

    def compute(b):
        @plsc.parallel_loop(0, _CHP, unroll=4)
        def _(rrow):
            for q in range(4):
                for j in range(0, _H, 16):
                    slg = (rrow * 4 + q, pl.ds(j, 16))
                    sle = (rrow, pl.ds(q * _H + j, 16))
                    gv[b][slg] = jnp.maximum(gv[b][slg] + ev[b][sle], 0.0)

    issue_inputs(my_base, 0)
    wait_inputs(my_base, 0)
    gather_copy(0).start()

    @pl.loop(0, 196)
    def _(kk):
        for b in (0, 1):
            k = kk * 2 + b
            ch = my_base + k

            @pl.when(k < my_n)
            def _():
                gather_copy(b).wait()

                @pl.when(k >= 1)
                def _():
                    scatter_copy(1 - b).wait()

                @pl.when(k + 1 < my_n)
                def _():
                    issue_inputs(ch + 1, 1 - b)

                compute(b)
                scatter_copy(b).start(add=True)

                @pl.when(k + 1 < my_n)
                def _():
                    wait_inputs(ch + 1, 1 - b)
                    gather_copy(1 - b).start()

    @pl.when(my_n == 391)
    def _():
        scatter_copy(0).wait()

    @pl.when(my_n == 390)
    def _():
        scatter_copy(1).wait()

    plsc.subcore_barrier()
    _acc_flush(acc_sh, out_hbm.at[c], s)


def _msg_sc(g2, eaw2, row, col, zeros_h):
    kfn = pl.kernel(
        _msg_sc_body,
        out_type=_f32(_NC, _N, _H),
        mesh=_sc_mesh(),
        compiler_params=_sc_params(),
        scratch_types=[
            pltpu.VMEM((_CH,), jnp.int32),
            pltpu.VMEM((_CH,), jnp.int32),
            pltpu.VMEM((_CH,), jnp.int32),
            pltpu.VMEM((_CH,), jnp.int32),
            pltpu.VMEM((_CH, _H), _F32),
            pltpu.VMEM((_CH, _H), _F32),
            pltpu.VMEM((_CHP, 128), _F32),
            pltpu.VMEM((_CHP, 128), _F32),
            pltpu.SemaphoreType.DMA,
            pltpu.SemaphoreType.DMA,
            pltpu.SemaphoreType.DMA,
            pltpu.SemaphoreType.DMA,
            pltpu.SemaphoreType.DMA,
            pltpu.SemaphoreType.DMA,
            pltpu.VMEM_SHARED((_N, _H), _F32),
        ],
    )
    return kfn(g2, eaw2, row, col, zeros_h)


def kernel(x, edge_index, edge_attr, batch, params):
    row = edge_index[0]
    col = edge_index[1]

    ones8 = jnp.ones((_CH, 8), _F32)
    zeros8 = jnp.zeros((_ZB, 8), _F32)
    zeros_h = jnp.zeros((_ZB, _H), _F32)

    h = _node_embed(x, params['x_emb_W'], params['x_emb_b'])
    ea2 = _edge_embed(edge_attr, params['edge_emb_W'], params['edge_emb_b'])

    degpart = _deg_sc(row, ones8, zeros8)
    dis, r, disw = _degpost(degpart)

    eaw2 = _prep_sc(disw, ea2, row)

    for l in range(_NL):
        lp = params['layers'][l]
        g2, st = _layer_pre(h, dis, r, lp['lin_W'], lp['lin_b'], lp['root'])
        agg = _msg_sc(g2, eaw2, row, col, zeros_h)
        z, stats = _layer_post(agg, dis, st, h)
        h = _bn_apply(z, stats, lp['bn_g'], lp['bn_b'], relu=(l < _NL - 1))

    sums, cnt = _pool(h, batch)
    return _mlp(sums, cnt, params['pred'])


# speedup vs baseline: 7.1445x; 1.3025x over previous
import functools

import jax
import jax.numpy as jnp
from jax import lax
from jax.experimental import pallas as pl
from jax.experimental.pallas import tpu as pltpu
from jax.experimental.pallas import tpu_sc as plsc

_N = 50000
_E = 800000
_EMB = 64
_G = 128
_NL = 5
_H = 32
_NC = 2
_NS = 16
_CH = 128
_CHP = _CH // 4
_NCHUNK = _E // _CH
_ZB = 3128
_ZT = _N - (_NS - 1) * _ZB

_F32 = jnp.float32


def _f32(*shape):
    return jax.ShapeDtypeStruct(shape, _F32)


def _matmul_bias_kernel(x_ref, w_ref, b_ref, o_ref):
    o_ref[...] = (
        jnp.dot(x_ref[...], w_ref[...].T, preferred_element_type=_F32)
        + b_ref[...]
    )


def _node_embed(x, w, b):
    nb = 1000
    return pl.pallas_call(
        _matmul_bias_kernel,
        grid=(_N // nb,),
        in_specs=[
            pl.BlockSpec((nb, 40), lambda i: (i, 0)),
            pl.BlockSpec((_EMB, 40), lambda i: (0, 0)),
            pl.BlockSpec((1, _EMB), lambda i: (0, 0)),
        ],
        out_specs=pl.BlockSpec((nb, _EMB), lambda i: (i, 0)),
        out_shape=_f32(_N, _EMB),
    )(x, w, b.reshape(1, _EMB))


def _edge_embed_kernel(a_ref, w_ref, b_ref, o_ref):
    a4 = a_ref[...]
    for half in range(2):
        wct = w_ref[...][half * _H:(half + 1) * _H, :].T
        cols = []
        for q in range(4):
            z_pre = jnp.zeros((10, q * _H), _F32)
            z_post = jnp.zeros((10, (3 - q) * _H), _F32)
            cols.append(jnp.concatenate(
                [x for x in (z_pre, wct, z_post) if x.shape[1]], axis=1))
        bd = jnp.concatenate(cols, axis=0)
        bias = jnp.concatenate([b_ref[...][:, half * _H:(half + 1) * _H]] * 4,
                               axis=1)
        o_ref[half] = (
            jnp.dot(a4, bd, preferred_element_type=_F32) + bias
        )


def _edge_embed(edge_attr, w, b):
    eb = 4000
    attr4 = edge_attr.reshape(_E // 4, 40)
    return pl.pallas_call(
        _edge_embed_kernel,
        grid=(_E // eb,),
        in_specs=[
            pl.BlockSpec((eb // 4, 40), lambda i: (i, 0)),
            pl.BlockSpec((_EMB, 10), lambda i: (0, 0)),
            pl.BlockSpec((1, _EMB), lambda i: (0, 0)),
        ],
        out_specs=pl.BlockSpec((2, eb // 4, 128), lambda i: (0, i, 0)),
        out_shape=_f32(2, _E // 4, 128),
    )(attr4, w, b.reshape(1, _EMB))


def _degpost_kernel(dp_ref, dis_ref, r_ref, disw_ref):
    d = dp_ref[0, :, 0:1] + dp_ref[1, :, 0:1] + 1.0
    dis = lax.rsqrt(d)
    dis_ref[...] = dis
    r_ref[...] = 1.0 / d
    disw_ref[...] = jnp.broadcast_to(dis, (dis.shape[0], _H))


def _degpost(degpart):
    nb = 1000
    return pl.pallas_call(
        _degpost_kernel,
        grid=(_N // nb,),
        in_specs=[pl.BlockSpec((2, nb, 8), lambda i: (0, i, 0))],
        out_specs=[
            pl.BlockSpec((nb, 1), lambda i: (i, 0)),
            pl.BlockSpec((nb, 1), lambda i: (i, 0)),
            pl.BlockSpec((nb, _H), lambda i: (i, 0)),
        ],
        out_shape=[_f32(_N, 1), _f32(_N, 1), _f32(_N, _H)],
    )(degpart)


def _pre_kernel(h_ref, dis_ref, r_ref, w_ref, b_ref, root_ref, g2_ref, st_ref):
    hp = (
        jnp.dot(h_ref[...], w_ref[...].T, preferred_element_type=_F32)
        + b_ref[...]
    )
    g = hp * dis_ref[...]
    g2_ref[0] = g[:, :_H]
    g2_ref[1] = g[:, _H:]
    st_ref[...] = jnp.maximum(hp + root_ref[...], 0.0) * r_ref[...]


def _layer_pre(h, dis, r, w, b, root):
    nb = 1000
    return pl.pallas_call(
        _pre_kernel,
        grid=(_N // nb,),
        in_specs=[
            pl.BlockSpec((nb, _EMB), lambda i: (i, 0)),
            pl.BlockSpec((nb, 1), lambda i: (i, 0)),
            pl.BlockSpec((nb, 1), lambda i: (i, 0)),
            pl.BlockSpec((_EMB, _EMB), lambda i: (0, 0)),
            pl.BlockSpec((1, _EMB), lambda i: (0, 0)),
            pl.BlockSpec((1, _EMB), lambda i: (0, 0)),
        ],
        out_specs=[
            pl.BlockSpec((2, nb, _H), lambda i: (0, i, 0)),
            pl.BlockSpec((nb, _EMB), lambda i: (i, 0)),
        ],
        out_shape=[_f32(2, _N, _H), _f32(_N, _EMB)],
    )(h, dis, r, w, b.reshape(1, _EMB), root)


def _post_kernel(agg_ref, dis_ref, st_ref, hin_ref, z_ref, stats_ref):
    aggf = jnp.concatenate([agg_ref[0], agg_ref[1]], axis=1)
    z = hin_ref[...] + dis_ref[...] * aggf + st_ref[...]
    z_ref[...] = z
    blockstats = jnp.concatenate(
        [jnp.sum(z, axis=0, keepdims=True),
         jnp.sum(z * z, axis=0, keepdims=True)], axis=0)

    @pl.when(pl.program_id(0) == 0)
    def _():
        stats_ref[...] = blockstats

    @pl.when(pl.program_id(0) != 0)
    def _():
        stats_ref[...] = stats_ref[...] + blockstats


def _layer_post(agg, dis, st, hin):
    nb = 1000
    return pl.pallas_call(
        _post_kernel,
        grid=(_N // nb,),
        in_specs=[
            pl.BlockSpec((2, nb, _H), lambda i: (0, i, 0)),
            pl.BlockSpec((nb, 1), lambda i: (i, 0)),
            pl.BlockSpec((nb, _EMB), lambda i: (i, 0)),
            pl.BlockSpec((nb, _EMB), lambda i: (i, 0)),
        ],
        out_specs=[
            pl.BlockSpec((nb, _EMB), lambda i: (i, 0)),
            pl.BlockSpec((2, _EMB), lambda i: (0, 0)),
        ],
        out_shape=[_f32(_N, _EMB), _f32(2, _EMB)],
    )(agg, dis, st, hin)


def _bn_kernel(z_ref, stats_ref, g_ref, b_ref, o_ref, *, relu):
    mu = stats_ref[0:1, :] * (1.0 / _N)
    var = stats_ref[1:2, :] * (1.0 / _N) - mu * mu
    inv = lax.rsqrt(var + 1e-5)
    h = (z_ref[...] - mu) * inv * g_ref[...] + b_ref[...]
    if relu:
        h = jnp.maximum(h, 0.0)
    o_ref[...] = h


def _bn_apply(z, stats, g, b, relu):
    nb = 1000
    return pl.pallas_call(
        functools.partial(_bn_kernel, relu=relu),
        grid=(_N // nb,),
        in_specs=[
            pl.BlockSpec((nb, _EMB), lambda i: (i, 0)),
            pl.BlockSpec((2, _EMB), lambda i: (0, 0)),
            pl.BlockSpec((1, _EMB), lambda i: (0, 0)),
            pl.BlockSpec((1, _EMB), lambda i: (0, 0)),
        ],
        out_specs=pl.BlockSpec((nb, _EMB), lambda i: (i, 0)),
        out_shape=_f32(_N, _EMB),
    )(z, stats, g.reshape(1, _EMB), b.reshape(1, _EMB))


def _pool_kernel(h_ref, b3_ref, sums_ref, cnt_ref):
    ids = b3_ref[0, 0, :]
    io = lax.broadcasted_iota(jnp.int32, (_G, ids.shape[0]), 0)
    oh = (io == ids[None, :]).astype(_F32)
    ps = jnp.dot(oh, h_ref[...], preferred_element_type=_F32)
    pc = jnp.sum(oh, axis=1, keepdims=True)

    @pl.when(pl.program_id(0) == 0)
    def _():
        sums_ref[...] = ps
        cnt_ref[...] = pc

    @pl.when(pl.program_id(0) != 0)
    def _():
        sums_ref[...] = sums_ref[...] + ps
        cnt_ref[...] = cnt_ref[...] + pc


def _pool(h, batch):
    nb = 1000
    batch3 = batch.reshape(_N // nb, 1, nb)
    return pl.pallas_call(
        _pool_kernel,
        grid=(_N // nb,),
        in_specs=[
            pl.BlockSpec((nb, _EMB), lambda i: (i, 0)),
            pl.BlockSpec((1, 1, nb), lambda i: (i, 0, 0)),
        ],
        out_specs=[
            pl.BlockSpec((_G, _EMB), lambda i: (0, 0)),
            pl.BlockSpec((_G, 1), lambda i: (0, 0)),
        ],
        out_shape=[_f32(_G, _EMB), _f32(_G, 1)],
    )(h, batch3)


def _mlp_kernel(s_ref, c_ref, w1_ref, b1_ref, w2_ref, b2_ref, w3_ref, b3_ref,
                o_ref):
    hg = s_ref[...] / jnp.maximum(c_ref[...], 1.0)
    z = jnp.maximum(
        jnp.dot(hg, w1_ref[...].T, preferred_element_type=_F32) + b1_ref[...],
        0.0)
    z = jnp.maximum(
        jnp.dot(z, w2_ref[...].T, preferred_element_type=_F32) + b2_ref[...],
        0.0)
    o_ref[...] = (
        jnp.sum(z * w3_ref[...], axis=1, keepdims=True) + b3_ref[...]
    )


def _mlp(sums, cnt, pred):
    (w1, b1), (w2, b2), (w3, b3) = pred
    hh = w1.shape[0]
    return pl.pallas_call(
        _mlp_kernel,
        out_shape=_f32(_G, 1),
    )(sums, cnt, w1, b1.reshape(1, hh), w2, b2.reshape(1, hh),
      w3, b3.reshape(1, 1))


def _sc_mesh():
    return plsc.VectorSubcoreMesh(core_axis_name="c", subcore_axis_name="s")


def _sc_params():
    return pltpu.CompilerParams(use_tc_tiling_on_sc=False)


def _acc_init(zeros_hbm, acc_sh, s):
    @pl.when(s < _NS - 1)
    def _():
        pltpu.sync_copy(zeros_hbm, acc_sh.at[pl.ds(s * _ZB, _ZB)])

    @pl.when(s == _NS - 1)
    def _():
        pltpu.sync_copy(zeros_hbm.at[pl.ds(0, _ZT)],
                        acc_sh.at[pl.ds((_NS - 1) * _ZB, _ZT)])


def _acc_flush(acc_sh, out_c, s):
    @pl.when(s < _NS - 1)
    def _():
        pltpu.sync_copy(acc_sh.at[pl.ds(s * _ZB, _ZB)],
                        out_c.at[pl.ds(s * _ZB, _ZB)])

    @pl.when(s == _NS - 1)
    def _():
        pltpu.sync_copy(acc_sh.at[pl.ds((_NS - 1) * _ZB, _ZT)],
                        out_c.at[pl.ds((_NS - 1) * _ZB, _ZT)])


def _deg_sc_body(row_hbm, ones_hbm, zeros_hbm, out_hbm, idx0, idx1, ones_v,
                 semi0, semi1, sems0, sems1, acc_sh):
    c = lax.axis_index("c")
    s = lax.axis_index("s")
    _acc_init(zeros_hbm, acc_sh, s)
    pltpu.sync_copy(ones_hbm, ones_v)
    plsc.subcore_barrier()
    my_n = 195 + (s < 5).astype(jnp.int32)
    my_base = c * (_NCHUNK // _NC) + s * 195 + jnp.minimum(s, 5)

    idx = (idx0, idx1)
    semi = (semi0, semi1)
    sems = (sems0, sems1)

    def idx_copy(ch, b):
        return pltpu.make_async_copy(row_hbm.at[pl.ds(ch * _CH, _CH)],
                                     idx[b], semi[b])

    def scatter_copy(b):
        return pltpu.make_async_copy(ones_v, acc_sh.at[idx[b]], sems[b])

    idx_copy(my_base, 0).start()
    idx_copy(my_base, 0).wait()

    @pl.loop(0, 98)
    def _(kk):
        for b in (0, 1):
            k = kk * 2 + b
            ch = my_base + k

            @pl.when(k < my_n)
            def _():
                @pl.when(k >= 1)
                def _():
                    scatter_copy(1 - b).wait()

                @pl.when(k + 1 < my_n)
                def _():
                    idx_copy(ch + 1, 1 - b).start()

                scatter_copy(b).start(add=True)

                @pl.when(k + 1 < my_n)
                def _():
                    idx_copy(ch + 1, 1 - b).wait()

    @pl.when(my_n == 196)
    def _():
        scatter_copy(1).wait()

    @pl.when(my_n == 195)
    def _():
        scatter_copy(0).wait()

    plsc.subcore_barrier()
    _acc_flush(acc_sh, out_hbm.at[c], s)


def _deg_sc(row, ones8, zeros8):
    kfn = pl.kernel(
        _deg_sc_body,
        out_type=_f32(_NC, _N, 8),
        mesh=_sc_mesh(),
        compiler_params=_sc_params(),
        scratch_types=[
            pltpu.VMEM((_CH,), jnp.int32),
            pltpu.VMEM((_CH,), jnp.int32),
            pltpu.VMEM((_CH, 8), _F32),
            pltpu.SemaphoreType.DMA,
            pltpu.SemaphoreType.DMA,
            pltpu.SemaphoreType.DMA,
            pltpu.SemaphoreType.DMA,
            pltpu.VMEM_SHARED((_N, 8), _F32),
        ],
    )
    return kfn(row, ones8, zeros8)


def _edge_split(s):
    my_n = 390 + (s < 10).astype(jnp.int32)
    my_base = s * 390 + jnp.minimum(s, 10)
    return my_n, my_base


def _prep_sc_body(disw_hbm, ea_hbm, row_hbm, out_hbm,
                  idx0, idx1, d0, d1, e0, e1,
                  semi0, semi1, semg0, semg1, semo0, semo1):
    c = lax.axis_index("c")
    s = lax.axis_index("s")
    my_n, my_base = _edge_split(s)

    idx = (idx0, idx1)
    dv = (d0, d1)
    ev = (e0, e1)
    semi = (semi0, semi1)
    semg = (semg0, semg1)
    semo = (semo0, semo1)

    def in_copies(ch, b):
        return (
            pltpu.make_async_copy(row_hbm.at[pl.ds(ch * _CH, _CH)],
                                  idx[b], semi[b]),
            pltpu.make_async_copy(ea_hbm.at[c].at[pl.ds(ch * _CHP, _CHP)],
                                  ev[b], semi[b]),
        )

    def gather_copy(b):
        return pltpu.make_async_copy(disw_hbm.at[idx[b]], dv[b], semg[b])

    def out_copy(ch, b):
        off = pl.ds(ch * _CHP, _CHP)
        return pltpu.make_async_copy(ev[b], out_hbm.at[c].at[off], semo[b])

    def compute(b):
        @plsc.parallel_loop(0, _CHP, unroll=4)
        def _(rrow):
            for q in range(4):
                for j in range(0, _H, 16):
                    sle = (rrow, pl.ds(q * _H + j, 16))
                    sld = (rrow * 4 + q, pl.ds(j, 16))
                    ev[b][sle] = dv[b][sld] * ev[b][sle]

    for cp in in_copies(my_base, 0):
        cp.start()
    for cp in in_copies(my_base, 0):
        cp.wait()
    gather_copy(0).start()

    @pl.loop(0, 196)
    def _(kk):
        for b in (0, 1):
            k = kk * 2 + b
            ch = my_base + k

            @pl.when(k < my_n)
            def _():
                gather_copy(b).wait()

                @pl.when(k >= 1)
                def _():
                    out_copy(ch - 1, 1 - b).wait()

                @pl.when(k + 1 < my_n)
                def _():
                    for cp in in_copies(ch + 1, 1 - b):
                        cp.start()

                compute(b)
                out_copy(ch, b).start()

                @pl.when(k + 1 < my_n)
                def _():
                    for cp in in_copies(ch + 1, 1 - b):
                        cp.wait()
                    gather_copy(1 - b).start()

    @pl.when(my_n == 391)
    def _():
        out_copy(my_base + 390, 0).wait()

    @pl.when(my_n == 390)
    def _():
        out_copy(my_base + 389, 1).wait()


def _prep_sc(disw, ea2, row):
    kfn = pl.kernel(
        _prep_sc_body,
        out_type=_f32(_NC, _E // 4, 128),
        mesh=_sc_mesh(),
        compiler_params=_sc_params(),
        scratch_types=[
            pltpu.VMEM((_CH,), jnp.int32),
            pltpu.VMEM((_CH,), jnp.int32),
            pltpu.VMEM((_CH, _H), _F32),
            pltpu.VMEM((_CH, _H), _F32),
            pltpu.VMEM((_CHP, 128), _F32),
            pltpu.VMEM((_CHP, 128), _F32),
            pltpu.SemaphoreType.DMA,
            pltpu.SemaphoreType.DMA,
            pltpu.SemaphoreType.DMA,
            pltpu.SemaphoreType.DMA,
            pltpu.SemaphoreType.DMA,
            pltpu.SemaphoreType.DMA,
        ],
    )
    return kfn(disw, ea2, row)


def _msg_sc_body(g2_hbm, eaw_hbm, row_hbm, col_hbm, zeros_hbm, out_hbm,
                 idxr0, idxr1, idxr2, idxc0, idxc1, idxc2,
                 g0, g1, g2v, e0, e1, e2,
                 semi0, semi1, semi2, semg0, semg1, semg2,
                 sems0, sems1, sems2, acc_sh):
    c = lax.axis_index("c")
    s = lax.axis_index("s")
    _acc_init(zeros_hbm, acc_sh, s)
    plsc.subcore_barrier()
    my_n, my_base = _edge_split(s)

    idxr = (idxr0, idxr1, idxr2)
    idxc = (idxc0, idxc1, idxc2)
    gv = (g0, g1, g2v)
    ev = (e0, e1, e2)
    semi = (semi0, semi1, semi2)
    semg = (semg0, semg1, semg2)
    sems = (sems0, sems1, sems2)

    def in_copies(ch, b):
        off = pl.ds(ch * _CH, _CH)
        return (
            pltpu.make_async_copy(row_hbm.at[off], idxr[b], semi[b]),
            pltpu.make_async_copy(col_hbm.at[off], idxc[b], semi[b]),
            pltpu.make_async_copy(eaw_hbm.at[c].at[pl.ds(ch * _CHP, _CHP)],
                                  ev[b], semi[b]),
        )

    def issue_inputs(ch, b):
        for cp in in_copies(ch, b):
            cp.start()

    def wait_inputs(ch, b):
        for cp in in_copies(ch, b):
            cp.wait()

    def gather_copy(b):
        return pltpu.make_async_copy(g2_hbm.at[c].at[idxr[b]], gv[b], semg[b])

    def scatter_copy(b):
        return pltpu.make_async_copy(gv[b], acc_sh.at[idxc[b]], sems[b])

    def compute(b):
        @plsc.parallel_loop(0, _CHP, unroll=4)
        def _(rrow):
            for q in range(4):
                for j in range(0, _H, 16):
                    slg = (rrow * 4 + q, pl.ds(j, 16))
                    sle = (rrow, pl.ds(q * _H + j, 16))
                    gv[b][slg] = jnp.maximum(gv[b][slg] + ev[b][sle], 0.0)

    issue_inputs(my_base, 0)
    issue_inputs(my_base + 1, 1)
    wait_inputs(my_base, 0)
    gather_copy(0).start()

    @pl.loop(0, 131)
    def _(kk):
        for d in (0, 1, 2):
            k = kk * 3 + d
            b = d
            ch = my_base + k
            bn = (d + 1) % 3
            bp = (d + 2) % 3

            @pl.when(k < my_n)
            def _():
                gather_copy(b).wait()

                @pl.when(k + 1 < my_n)
                def _():
                    wait_inputs(ch + 1, bn)
                    gather_copy(bn).start()

                @pl.when(k >= 1)
                def _():
                    scatter_copy(bp).wait()

                @pl.when(k + 2 < my_n)
                def _():
                    issue_inputs(ch + 2, bp)

                compute(b)
                scatter_copy(b).start(add=True)

    @pl.when(my_n == 391)
    def _():
        scatter_copy(390 % 3).wait()

    @pl.when(my_n == 390)
    def _():
        scatter_copy(389 % 3).wait()

    plsc.subcore_barrier()
    _acc_flush(acc_sh, out_hbm.at[c], s)


def _msg_sc(g2, eaw2, row, col, zeros_h):
    kfn = pl.kernel(
        _msg_sc_body,
        out_type=_f32(_NC, _N, _H),
        mesh=_sc_mesh(),
        compiler_params=_sc_params(),
        scratch_types=[
            pltpu.VMEM((_CH,), jnp.int32),
            pltpu.VMEM((_CH,), jnp.int32),
            pltpu.VMEM((_CH,), jnp.int32),
            pltpu.VMEM((_CH,), jnp.int32),
            pltpu.VMEM((_CH,), jnp.int32),
            pltpu.VMEM((_CH,), jnp.int32),
            pltpu.VMEM((_CH, _H), _F32),
            pltpu.VMEM((_CH, _H), _F32),
            pltpu.VMEM((_CH, _H), _F32),
            pltpu.VMEM((_CHP, 128), _F32),
            pltpu.VMEM((_CHP, 128), _F32),
            pltpu.VMEM((_CHP, 128), _F32),
            pltpu.SemaphoreType.DMA,
            pltpu.SemaphoreType.DMA,
            pltpu.SemaphoreType.DMA,
            pltpu.SemaphoreType.DMA,
            pltpu.SemaphoreType.DMA,
            pltpu.SemaphoreType.DMA,
            pltpu.SemaphoreType.DMA,
            pltpu.SemaphoreType.DMA,
            pltpu.SemaphoreType.DMA,
            pltpu.VMEM_SHARED((_N, _H), _F32),
        ],
    )
    return kfn(g2, eaw2, row, col, zeros_h)


def kernel(x, edge_index, edge_attr, batch, params):
    row = edge_index[0]
    col = edge_index[1]

    ones8 = jnp.ones((_CH, 8), _F32)
    zeros8 = jnp.zeros((_ZB, 8), _F32)
    zeros_h = jnp.zeros((_ZB, _H), _F32)

    h = _node_embed(x, params['x_emb_W'], params['x_emb_b'])
    ea2 = _edge_embed(edge_attr, params['edge_emb_W'], params['edge_emb_b'])

    degpart = _deg_sc(row, ones8, zeros8)
    dis, r, disw = _degpost(degpart)

    eaw2 = _prep_sc(disw, ea2, row)

    for l in range(_NL):
        lp = params['layers'][l]
        g2, st = _layer_pre(h, dis, r, lp['lin_W'], lp['lin_b'], lp['root'])
        agg = _msg_sc(g2, eaw2, row, col, zeros_h)
        z, stats = _layer_post(agg, dis, st, h)
        h = _bn_apply(z, stats, lp['bn_g'], lp['bn_b'], relu=(l < _NL - 1))

    sums, cnt = _pool(h, batch)
    return _mlp(sums, cnt, params['pred'])


# speedup vs baseline: 7.6086x; 1.0650x over previous
import functools

import jax
import jax.numpy as jnp
from jax import lax
from jax.experimental import pallas as pl
from jax.experimental.pallas import tpu as pltpu
from jax.experimental.pallas import tpu_sc as plsc

_N = 50000
_E = 800000
_EMB = 64
_G = 128
_NL = 5
_H = 32
_NC = 2
_NS = 16
_CH = 128
_CHP = _CH // 4
_NCHUNK = _E // _CH
_ZB = 3128
_ZT = _N - (_NS - 1) * _ZB

_F32 = jnp.float32


def _f32(*shape):
    return jax.ShapeDtypeStruct(shape, _F32)


def _matmul_bias_kernel(x_ref, w_ref, b_ref, o_ref):
    o_ref[...] = (
        jnp.dot(x_ref[...], w_ref[...].T, preferred_element_type=_F32)
        + b_ref[...]
    )


def _node_embed(x, w, b):
    nb = 1000
    return pl.pallas_call(
        _matmul_bias_kernel,
        grid=(_N // nb,),
        in_specs=[
            pl.BlockSpec((nb, 40), lambda i: (i, 0)),
            pl.BlockSpec((_EMB, 40), lambda i: (0, 0)),
            pl.BlockSpec((1, _EMB), lambda i: (0, 0)),
        ],
        out_specs=pl.BlockSpec((nb, _EMB), lambda i: (i, 0)),
        out_shape=_f32(_N, _EMB),
    )(x, w, b.reshape(1, _EMB))


def _edge_embed_kernel(a_ref, w_ref, b_ref, o_ref):
    a4 = a_ref[...]
    for half in range(2):
        wct = w_ref[...][half * _H:(half + 1) * _H, :].T
        cols = []
        for q in range(4):
            z_pre = jnp.zeros((10, q * _H), _F32)
            z_post = jnp.zeros((10, (3 - q) * _H), _F32)
            cols.append(jnp.concatenate(
                [x for x in (z_pre, wct, z_post) if x.shape[1]], axis=1))
        bd = jnp.concatenate(cols, axis=0)
        bias = jnp.concatenate([b_ref[...][:, half * _H:(half + 1) * _H]] * 4,
                               axis=1)
        o_ref[half] = (
            jnp.dot(a4, bd, preferred_element_type=_F32) + bias
        )


def _edge_embed(edge_attr, w, b):
    eb = 4000
    attr4 = edge_attr.reshape(_E // 4, 40)
    return pl.pallas_call(
        _edge_embed_kernel,
        grid=(_E // eb,),
        in_specs=[
            pl.BlockSpec((eb // 4, 40), lambda i: (i, 0)),
            pl.BlockSpec((_EMB, 10), lambda i: (0, 0)),
            pl.BlockSpec((1, _EMB), lambda i: (0, 0)),
        ],
        out_specs=pl.BlockSpec((2, eb // 4, 128), lambda i: (0, i, 0)),
        out_shape=_f32(2, _E // 4, 128),
    )(attr4, w, b.reshape(1, _EMB))


def _degpost_kernel(dp_ref, dis_ref, r_ref, disw_ref):
    d = dp_ref[0, :, 0:1] + dp_ref[1, :, 0:1] + 1.0
    dis = lax.rsqrt(d)
    dis_ref[...] = dis
    r_ref[...] = 1.0 / d
    disw_ref[...] = jnp.broadcast_to(dis, (dis.shape[0], _H))


def _degpost(degpart):
    nb = 1000
    return pl.pallas_call(
        _degpost_kernel,
        grid=(_N // nb,),
        in_specs=[pl.BlockSpec((2, nb, 8), lambda i: (0, i, 0))],
        out_specs=[
            pl.BlockSpec((nb, 1), lambda i: (i, 0)),
            pl.BlockSpec((nb, 1), lambda i: (i, 0)),
            pl.BlockSpec((nb, _H), lambda i: (i, 0)),
        ],
        out_shape=[_f32(_N, 1), _f32(_N, 1), _f32(_N, _H)],
    )(degpart)


def _pre_kernel(h_ref, dis_ref, r_ref, w_ref, b_ref, root_ref, g2_ref, st_ref):
    hp = (
        jnp.dot(h_ref[...], w_ref[...].T, preferred_element_type=_F32)
        + b_ref[...]
    )
    g = hp * dis_ref[...]
    g2_ref[0] = g[:, :_H]
    g2_ref[1] = g[:, _H:]
    st_ref[...] = jnp.maximum(hp + root_ref[...], 0.0) * r_ref[...]


def _layer_pre(h, dis, r, w, b, root):
    nb = 1000
    return pl.pallas_call(
        _pre_kernel,
        grid=(_N // nb,),
        in_specs=[
            pl.BlockSpec((nb, _EMB), lambda i: (i, 0)),
            pl.BlockSpec((nb, 1), lambda i: (i, 0)),
            pl.BlockSpec((nb, 1), lambda i: (i, 0)),
            pl.BlockSpec((_EMB, _EMB), lambda i: (0, 0)),
            pl.BlockSpec((1, _EMB), lambda i: (0, 0)),
            pl.BlockSpec((1, _EMB), lambda i: (0, 0)),
        ],
        out_specs=[
            pl.BlockSpec((2, nb, _H), lambda i: (0, i, 0)),
            pl.BlockSpec((nb, _EMB), lambda i: (i, 0)),
        ],
        out_shape=[_f32(2, _N, _H), _f32(_N, _EMB)],
    )(h, dis, r, w, b.reshape(1, _EMB), root)


def _post_kernel(agg_ref, dis_ref, st_ref, hin_ref, z_ref, stats_ref):
    aggf = jnp.concatenate([agg_ref[0], agg_ref[1]], axis=1)
    z = hin_ref[...] + dis_ref[...] * aggf + st_ref[...]
    z_ref[...] = z
    blockstats = jnp.concatenate(
        [jnp.sum(z, axis=0, keepdims=True),
         jnp.sum(z * z, axis=0, keepdims=True)], axis=0)

    @pl.when(pl.program_id(0) == 0)
    def _():
        stats_ref[...] = blockstats

    @pl.when(pl.program_id(0) != 0)
    def _():
        stats_ref[...] = stats_ref[...] + blockstats


def _layer_post(agg, dis, st, hin):
    nb = 1000
    return pl.pallas_call(
        _post_kernel,
        grid=(_N // nb,),
        in_specs=[
            pl.BlockSpec((2, nb, _H), lambda i: (0, i, 0)),
            pl.BlockSpec((nb, 1), lambda i: (i, 0)),
            pl.BlockSpec((nb, _EMB), lambda i: (i, 0)),
            pl.BlockSpec((nb, _EMB), lambda i: (i, 0)),
        ],
        out_specs=[
            pl.BlockSpec((nb, _EMB), lambda i: (i, 0)),
            pl.BlockSpec((2, _EMB), lambda i: (0, 0)),
        ],
        out_shape=[_f32(_N, _EMB), _f32(2, _EMB)],
    )(agg, dis, st, hin)


def _bn_kernel(z_ref, stats_ref, g_ref, b_ref, o_ref, *, relu):
    mu = stats_ref[0:1, :] * (1.0 / _N)
    var = stats_ref[1:2, :] * (1.0 / _N) - mu * mu
    inv = lax.rsqrt(var + 1e-5)
    h = (z_ref[...] - mu) * inv * g_ref[...] + b_ref[...]
    if relu:
        h = jnp.maximum(h, 0.0)
    o_ref[...] = h


def _bn_apply(z, stats, g, b, relu):
    nb = 1000
    return pl.pallas_call(
        functools.partial(_bn_kernel, relu=relu),
        grid=(_N // nb,),
        in_specs=[
            pl.BlockSpec((nb, _EMB), lambda i: (i, 0)),
            pl.BlockSpec((2, _EMB), lambda i: (0, 0)),
            pl.BlockSpec((1, _EMB), lambda i: (0, 0)),
            pl.BlockSpec((1, _EMB), lambda i: (0, 0)),
        ],
        out_specs=pl.BlockSpec((nb, _EMB), lambda i: (i, 0)),
        out_shape=_f32(_N, _EMB),
    )(z, stats, g.reshape(1, _EMB), b.reshape(1, _EMB))


def _pool_kernel(h_ref, b3_ref, sums_ref, cnt_ref):
    ids = b3_ref[0, 0, :]
    io = lax.broadcasted_iota(jnp.int32, (_G, ids.shape[0]), 0)
    oh = (io == ids[None, :]).astype(_F32)
    ps = jnp.dot(oh, h_ref[...], preferred_element_type=_F32)
    pc = jnp.sum(oh, axis=1, keepdims=True)

    @pl.when(pl.program_id(0) == 0)
    def _():
        sums_ref[...] = ps
        cnt_ref[...] = pc

    @pl.when(pl.program_id(0) != 0)
    def _():
        sums_ref[...] = sums_ref[...] + ps
        cnt_ref[...] = cnt_ref[...] + pc


def _pool(h, batch):
    nb = 1000
    batch3 = batch.reshape(_N // nb, 1, nb)
    return pl.pallas_call(
        _pool_kernel,
        grid=(_N // nb,),
        in_specs=[
            pl.BlockSpec((nb, _EMB), lambda i: (i, 0)),
            pl.BlockSpec((1, 1, nb), lambda i: (i, 0, 0)),
        ],
        out_specs=[
            pl.BlockSpec((_G, _EMB), lambda i: (0, 0)),
            pl.BlockSpec((_G, 1), lambda i: (0, 0)),
        ],
        out_shape=[_f32(_G, _EMB), _f32(_G, 1)],
    )(h, batch3)


def _mlp_kernel(s_ref, c_ref, w1_ref, b1_ref, w2_ref, b2_ref, w3_ref, b3_ref,
                o_ref):
    hg = s_ref[...] / jnp.maximum(c_ref[...], 1.0)
    z = jnp.maximum(
        jnp.dot(hg, w1_ref[...].T, preferred_element_type=_F32) + b1_ref[...],
        0.0)
    z = jnp.maximum(
        jnp.dot(z, w2_ref[...].T, preferred_element_type=_F32) + b2_ref[...],
        0.0)
    o_ref[...] = (
        jnp.sum(z * w3_ref[...], axis=1, keepdims=True) + b3_ref[...]
    )


def _mlp(sums, cnt, pred):
    (w1, b1), (w2, b2), (w3, b3) = pred
    hh = w1.shape[0]
    return pl.pallas_call(
        _mlp_kernel,
        out_shape=_f32(_G, 1),
    )(sums, cnt, w1, b1.reshape(1, hh), w2, b2.reshape(1, hh),
      w3, b3.reshape(1, 1))


def _sc_mesh():
    return plsc.VectorSubcoreMesh(core_axis_name="c", subcore_axis_name="s")


def _sc_params():
    return pltpu.CompilerParams(use_tc_tiling_on_sc=False)


def _acc_init(zeros_hbm, acc_sh, s):
    @pl.when(s < _NS - 1)
    def _():
        pltpu.sync_copy(zeros_hbm, acc_sh.at[pl.ds(s * _ZB, _ZB)])

    @pl.when(s == _NS - 1)
    def _():
        pltpu.sync_copy(zeros_hbm.at[pl.ds(0, _ZT)],
                        acc_sh.at[pl.ds((_NS - 1) * _ZB, _ZT)])


def _acc_flush(acc_sh, out_c, s):
    @pl.when(s < _NS - 1)
    def _():
        pltpu.sync_copy(acc_sh.at[pl.ds(s * _ZB, _ZB)],
                        out_c.at[pl.ds(s * _ZB, _ZB)])

    @pl.when(s == _NS - 1)
    def _():
        pltpu.sync_copy(acc_sh.at[pl.ds((_NS - 1) * _ZB, _ZT)],
                        out_c.at[pl.ds((_NS - 1) * _ZB, _ZT)])


def _deg_sc_body(row_hbm, ones_hbm, zeros_hbm, out_hbm, idx0, idx1, ones_v,
                 semi0, semi1, sems0, sems1, acc_sh):
    c = lax.axis_index("c")
    s = lax.axis_index("s")
    _acc_init(zeros_hbm, acc_sh, s)
    pltpu.sync_copy(ones_hbm, ones_v)
    plsc.subcore_barrier()
    my_n = 195 + (s < 5).astype(jnp.int32)
    my_base = c * (_NCHUNK // _NC) + s * 195 + jnp.minimum(s, 5)

    idx = (idx0, idx1)
    semi = (semi0, semi1)
    sems = (sems0, sems1)

    def idx_copy(ch, b):
        return pltpu.make_async_copy(row_hbm.at[pl.ds(ch * _CH, _CH)],
                                     idx[b], semi[b])

    def scatter_copy(b):
        return pltpu.make_async_copy(ones_v, acc_sh.at[idx[b]], sems[b])

    idx_copy(my_base, 0).start()
    idx_copy(my_base, 0).wait()

    @pl.loop(0, 98)
    def _(kk):
        for b in (0, 1):
            k = kk * 2 + b
            ch = my_base + k

            @pl.when(k < my_n)
            def _():
                @pl.when(k >= 1)
                def _():
                    scatter_copy(1 - b).wait()

                @pl.when(k + 1 < my_n)
                def _():
                    idx_copy(ch + 1, 1 - b).start()

                scatter_copy(b).start(add=True)

                @pl.when(k + 1 < my_n)
                def _():
                    idx_copy(ch + 1, 1 - b).wait()

    @pl.when(my_n == 196)
    def _():
        scatter_copy(1).wait()

    @pl.when(my_n == 195)
    def _():
        scatter_copy(0).wait()

    plsc.subcore_barrier()
    _acc_flush(acc_sh, out_hbm.at[c], s)


def _deg_sc(row, ones8, zeros8):
    kfn = pl.kernel(
        _deg_sc_body,
        out_type=_f32(_NC, _N, 8),
        mesh=_sc_mesh(),
        compiler_params=_sc_params(),
        scratch_types=[
            pltpu.VMEM((_CH,), jnp.int32),
            pltpu.VMEM((_CH,), jnp.int32),
            pltpu.VMEM((_CH, 8), _F32),
            pltpu.SemaphoreType.DMA,
            pltpu.SemaphoreType.DMA,
            pltpu.SemaphoreType.DMA,
            pltpu.SemaphoreType.DMA,
            pltpu.VMEM_SHARED((_N, 8), _F32),
        ],
    )
    return kfn(row, ones8, zeros8)


def _edge_split(s):
    my_n = 390 + (s < 10).astype(jnp.int32)
    my_base = s * 390 + jnp.minimum(s, 10)
    return my_n, my_base


def _prep_sc_body(disw_hbm, ea_hbm, row_hbm, out_hbm,
                  idx0, idx1, idx2, d0, d1, d2, e0, e1, e2,
                  semi0, semi1, semi2, semg0, semg1, semg2,
                  semo0, semo1, semo2):
    c = lax.axis_index("c")
    s = lax.axis_index("s")
    my_n, my_base = _edge_split(s)

    idx = (idx0, idx1, idx2)
    dv = (d0, d1, d2)
    ev = (e0, e1, e2)
    semi = (semi0, semi1, semi2)
    semg = (semg0, semg1, semg2)
    semo = (semo0, semo1, semo2)

    def in_copies(ch, b):
        return (
            pltpu.make_async_copy(row_hbm.at[pl.ds(ch * _CH, _CH)],
                                  idx[b], semi[b]),
            pltpu.make_async_copy(ea_hbm.at[c].at[pl.ds(ch * _CHP, _CHP)],
                                  ev[b], semi[b]),
        )

    def gather_copy(b):
        return pltpu.make_async_copy(disw_hbm.at[idx[b]], dv[b], semg[b])

    def out_copy(ch, b):
        off = pl.ds(ch * _CHP, _CHP)
        return pltpu.make_async_copy(ev[b], out_hbm.at[c].at[off], semo[b])

    def compute(b):
        @plsc.parallel_loop(0, _CHP, unroll=4)
        def _(rrow):
            for q in range(4):
                for j in range(0, _H, 16):
                    sle = (rrow, pl.ds(q * _H + j, 16))
                    sld = (rrow * 4 + q, pl.ds(j, 16))
                    ev[b][sle] = dv[b][sld] * ev[b][sle]

    for cp in in_copies(my_base, 0):
        cp.start()
    for cp in in_copies(my_base + 1, 1):
        cp.start()
    for cp in in_copies(my_base, 0):
        cp.wait()
    gather_copy(0).start()

    @pl.loop(0, 131)
    def _(kk):
        for d in (0, 1, 2):
            k = kk * 3 + d
            b = d
            ch = my_base + k
            bn = (d + 1) % 3
            bp = (d + 2) % 3

            @pl.when(k < my_n)
            def _():
                gather_copy(b).wait()

                @pl.when(k + 1 < my_n)
                def _():
                    for cp in in_copies(ch + 1, bn):
                        cp.wait()
                    gather_copy(bn).start()

                @pl.when(k >= 1)
                def _():
                    out_copy(ch - 1, bp).wait()

                @pl.when(k + 2 < my_n)
                def _():
                    for cp in in_copies(ch + 2, bp):
                        cp.start()

                compute(b)
                out_copy(ch, b).start()

    @pl.when(my_n == 391)
    def _():
        out_copy(my_base + 390, 390 % 3).wait()

    @pl.when(my_n == 390)
    def _():
        out_copy(my_base + 389, 389 % 3).wait()


def _prep_sc(disw, ea2, row):
    kfn = pl.kernel(
        _prep_sc_body,
        out_type=_f32(_NC, _E // 4, 128),
        mesh=_sc_mesh(),
        compiler_params=_sc_params(),
        scratch_types=[
            pltpu.VMEM((_CH,), jnp.int32),
            pltpu.VMEM((_CH,), jnp.int32),
            pltpu.VMEM((_CH,), jnp.int32),
            pltpu.VMEM((_CH, _H), _F32),
            pltpu.VMEM((_CH, _H), _F32),
            pltpu.VMEM((_CH, _H), _F32),
            pltpu.VMEM((_CHP, 128), _F32),
            pltpu.VMEM((_CHP, 128), _F32),
            pltpu.VMEM((_CHP, 128), _F32),
            pltpu.SemaphoreType.DMA,
            pltpu.SemaphoreType.DMA,
            pltpu.SemaphoreType.DMA,
            pltpu.SemaphoreType.DMA,
            pltpu.SemaphoreType.DMA,
            pltpu.SemaphoreType.DMA,
            pltpu.SemaphoreType.DMA,
            pltpu.SemaphoreType.DMA,
            pltpu.SemaphoreType.DMA,
        ],
    )
    return kfn(disw, ea2, row)


def _msg_sc_body(g2_hbm, eaw_hbm, row_hbm, col_hbm, zeros_hbm, out_hbm,
                 idxr0, idxr1, idxr2, idxc0, idxc1, idxc2,
                 g0, g1, g2v, e0, e1, e2,
                 semi0, semi1, semi2, semg0, semg1, semg2,
                 sems0, sems1, sems2, acc_sh):
    c = lax.axis_index("c")
    s = lax.axis_index("s")
    _acc_init(zeros_hbm, acc_sh, s)
    plsc.subcore_barrier()
    my_n, my_base = _edge_split(s)

    idxr = (idxr0, idxr1, idxr2)
    idxc = (idxc0, idxc1, idxc2)
    gv = (g0, g1, g2v)
    ev = (e0, e1, e2)
    semi = (semi0, semi1, semi2)
    semg = (semg0, semg1, semg2)
    sems = (sems0, sems1, sems2)

    def in_copies(ch, b):
        off = pl.ds(ch * _CH, _CH)
        return (
            pltpu.make_async_copy(row_hbm.at[off], idxr[b], semi[b]),
            pltpu.make_async_copy(col_hbm.at[off], idxc[b], semi[b]),
            pltpu.make_async_copy(eaw_hbm.at[c].at[pl.ds(ch * _CHP, _CHP)],
                                  ev[b], semi[b]),
        )

    def issue_inputs(ch, b):
        for cp in in_copies(ch, b):
            cp.start()

    def wait_inputs(ch, b):
        for cp in in_copies(ch, b):
            cp.wait()

    def gather_copy(b):
        return pltpu.make_async_copy(g2_hbm.at[c].at[idxr[b]], gv[b], semg[b])

    def scatter_copy(b):
        return pltpu.make_async_copy(gv[b], acc_sh.at[idxc[b]], sems[b])

    def compute(b):
        @plsc.parallel_loop(0, _CHP, unroll=4)
        def _(rrow):
            for q in range(4):
                for j in range(0, _H, 16):
                    slg = (rrow * 4 + q, pl.ds(j, 16))
                    sle = (rrow, pl.ds(q * _H + j, 16))
                    gv[b][slg] = jnp.maximum(gv[b][slg] + ev[b][sle], 0.0)

    issue_inputs(my_base, 0)
    issue_inputs(my_base + 1, 1)
    wait_inputs(my_base, 0)
    gather_copy(0).start()

    @pl.loop(0, 131)
    def _(kk):
        for d in (0, 1, 2):
            k = kk * 3 + d
            b = d
            ch = my_base + k
            bn = (d + 1) % 3
            bp = (d + 2) % 3

            @pl.when(k < my_n)
            def _():
                gather_copy(b).wait()

                @pl.when(k + 1 < my_n)
                def _():
                    wait_inputs(ch + 1, bn)
                    gather_copy(bn).start()

                @pl.when(k >= 1)
                def _():
                    scatter_copy(bp).wait()

                @pl.when(k + 2 < my_n)
                def _():
                    issue_inputs(ch + 2, bp)

                compute(b)
                scatter_copy(b).start(add=True)

    @pl.when(my_n == 391)
    def _():
        scatter_copy(390 % 3).wait()

    @pl.when(my_n == 390)
    def _():
        scatter_copy(389 % 3).wait()

    plsc.subcore_barrier()
    _acc_flush(acc_sh, out_hbm.at[c], s)


def _msg_sc(g2, eaw2, row, col, zeros_h):
    kfn = pl.kernel(
        _msg_sc_body,
        out_type=_f32(_NC, _N, _H),
        mesh=_sc_mesh(),
        compiler_params=_sc_params(),
        scratch_types=[
            pltpu.VMEM((_CH,), jnp.int32),
            pltpu.VMEM((_CH,), jnp.int32),
            pltpu.VMEM((_CH,), jnp.int32),
            pltpu.VMEM((_CH,), jnp.int32),
            pltpu.VMEM((_CH,), jnp.int32),
            pltpu.VMEM((_CH,), jnp.int32),
            pltpu.VMEM((_CH, _H), _F32),
            pltpu.VMEM((_CH, _H), _F32),
            pltpu.VMEM((_CH, _H), _F32),
            pltpu.VMEM((_CHP, 128), _F32),
            pltpu.VMEM((_CHP, 128), _F32),
            pltpu.VMEM((_CHP, 128), _F32),
            pltpu.SemaphoreType.DMA,
            pltpu.SemaphoreType.DMA,
            pltpu.SemaphoreType.DMA,
            pltpu.SemaphoreType.DMA,
            pltpu.SemaphoreType.DMA,
            pltpu.SemaphoreType.DMA,
            pltpu.SemaphoreType.DMA,
            pltpu.SemaphoreType.DMA,
            pltpu.SemaphoreType.DMA,
            pltpu.VMEM_SHARED((_N, _H), _F32),
        ],
    )
    return kfn(g2, eaw2, row, col, zeros_h)


def kernel(x, edge_index, edge_attr, batch, params):
    row = edge_index[0]
    col = edge_index[1]

    ones8 = jnp.ones((_CH, 8), _F32)
    zeros8 = jnp.zeros((_ZB, 8), _F32)
    zeros_h = jnp.zeros((_ZB, _H), _F32)

    h = _node_embed(x, params['x_emb_W'], params['x_emb_b'])
    ea2 = _edge_embed(edge_attr, params['edge_emb_W'], params['edge_emb_b'])

    degpart = _deg_sc(row, ones8, zeros8)
    dis, r, disw = _degpost(degpart)

    eaw2 = _prep_sc(disw, ea2, row)

    for l in range(_NL):
        lp = params['layers'][l]
        g2, st = _layer_pre(h, dis, r, lp['lin_W'], lp['lin_b'], lp['root'])
        agg = _msg_sc(g2, eaw2, row, col, zeros_h)
        z, stats = _layer_post(agg, dis, st, h)
        h = _bn_apply(z, stats, lp['bn_g'], lp['bn_b'], relu=(l < _NL - 1))

    sums, cnt = _pool(h, batch)
    return _mlp(sums, cnt, params['pred'])


# speedup vs baseline: 7.9393x; 1.0435x over previous
import functools

import jax
import jax.numpy as jnp
from jax import lax
from jax.experimental import pallas as pl
from jax.experimental.pallas import tpu as pltpu
from jax.experimental.pallas import tpu_sc as plsc

_N = 50000
_E = 800000
_EMB = 64
_G = 128
_NL = 5
_H = 32
_NC = 2
_NS = 16
_CH = 128
_CHP = _CH // 4
_NCHUNK = _E // _CH
_ZB = 3128
_ZT = _N - (_NS - 1) * _ZB

_F32 = jnp.float32


def _f32(*shape):
    return jax.ShapeDtypeStruct(shape, _F32)


def _matmul_bias_kernel(x_ref, w_ref, b_ref, o_ref):
    o_ref[...] = (
        jnp.dot(x_ref[...], w_ref[...].T, preferred_element_type=_F32)
        + b_ref[...]
    )


def _node_embed(x, w, b):
    nb = 1000
    return pl.pallas_call(
        _matmul_bias_kernel,
        grid=(_N // nb,),
        in_specs=[
            pl.BlockSpec((nb, 40), lambda i: (i, 0)),
            pl.BlockSpec((_EMB, 40), lambda i: (0, 0)),
            pl.BlockSpec((1, _EMB), lambda i: (0, 0)),
        ],
        out_specs=pl.BlockSpec((nb, _EMB), lambda i: (i, 0)),
        out_shape=_f32(_N, _EMB),
    )(x, w, b.reshape(1, _EMB))


def _edge_embed_kernel(a_ref, w_ref, b_ref, o_ref):
    a4 = a_ref[...]
    for half in range(2):
        wct = w_ref[...][half * _H:(half + 1) * _H, :].T
        cols = []
        for q in range(4):
            z_pre = jnp.zeros((10, q * _H), _F32)
            z_post = jnp.zeros((10, (3 - q) * _H), _F32)
            cols.append(jnp.concatenate(
                [x for x in (z_pre, wct, z_post) if x.shape[1]], axis=1))
        bd = jnp.concatenate(cols, axis=0)
        bias = jnp.concatenate([b_ref[...][:, half * _H:(half + 1) * _H]] * 4,
                               axis=1)
        o_ref[half] = (
            jnp.dot(a4, bd, preferred_element_type=_F32) + bias
        )


def _edge_embed(edge_attr, w, b):
    eb = 4000
    attr4 = edge_attr.reshape(_E // 4, 40)
    return pl.pallas_call(
        _edge_embed_kernel,
        grid=(_E // eb,),
        in_specs=[
            pl.BlockSpec((eb // 4, 40), lambda i: (i, 0)),
            pl.BlockSpec((_EMB, 10), lambda i: (0, 0)),
            pl.BlockSpec((1, _EMB), lambda i: (0, 0)),
        ],
        out_specs=pl.BlockSpec((2, eb // 4, 128), lambda i: (0, i, 0)),
        out_shape=_f32(2, _E // 4, 128),
    )(attr4, w, b.reshape(1, _EMB))


def _degpost_kernel(dp_ref, dis_ref, r_ref, disw_ref):
    d = dp_ref[0, :, 0:1] + dp_ref[1, :, 0:1] + 1.0
    dis = lax.rsqrt(d)
    dis_ref[...] = dis
    r_ref[...] = 1.0 / d
    disw_ref[...] = jnp.broadcast_to(dis, (dis.shape[0], _H))


def _degpost(degpart):
    nb = 1000
    return pl.pallas_call(
        _degpost_kernel,
        grid=(_N // nb,),
        in_specs=[pl.BlockSpec((2, nb, 8), lambda i: (0, i, 0))],
        out_specs=[
            pl.BlockSpec((nb, 1), lambda i: (i, 0)),
            pl.BlockSpec((nb, 1), lambda i: (i, 0)),
            pl.BlockSpec((nb, _H), lambda i: (i, 0)),
        ],
        out_shape=[_f32(_N, 1), _f32(_N, 1), _f32(_N, _H)],
    )(degpart)


def _pre_kernel(h_ref, dis_ref, r_ref, w_ref, b_ref, root_ref, g2_ref, st_ref):
    hp = (
        jnp.dot(h_ref[...], w_ref[...].T, preferred_element_type=_F32)
        + b_ref[...]
    )
    g = hp * dis_ref[...]
    g2_ref[0] = g[:, :_H]
    g2_ref[1] = g[:, _H:]
    st_ref[...] = jnp.maximum(hp + root_ref[...], 0.0) * r_ref[...]


def _layer_pre(h, dis, r, w, b, root):
    nb = 1000
    return pl.pallas_call(
        _pre_kernel,
        grid=(_N // nb,),
        in_specs=[
            pl.BlockSpec((nb, _EMB), lambda i: (i, 0)),
            pl.BlockSpec((nb, 1), lambda i: (i, 0)),
            pl.BlockSpec((nb, 1), lambda i: (i, 0)),
            pl.BlockSpec((_EMB, _EMB), lambda i: (0, 0)),
            pl.BlockSpec((1, _EMB), lambda i: (0, 0)),
            pl.BlockSpec((1, _EMB), lambda i: (0, 0)),
        ],
        out_specs=[
            pl.BlockSpec((2, nb, _H), lambda i: (0, i, 0)),
            pl.BlockSpec((nb, _EMB), lambda i: (i, 0)),
        ],
        out_shape=[_f32(2, _N, _H), _f32(_N, _EMB)],
    )(h, dis, r, w, b.reshape(1, _EMB), root)


def _post_kernel(agg_ref, dis_ref, st_ref, hin_ref, z_ref, stats_ref):
    aggf = jnp.concatenate([agg_ref[0], agg_ref[1]], axis=1)
    z = hin_ref[...] + dis_ref[...] * aggf + st_ref[...]
    z_ref[...] = z
    blockstats = jnp.concatenate(
        [jnp.sum(z, axis=0, keepdims=True),
         jnp.sum(z * z, axis=0, keepdims=True)], axis=0)

    @pl.when(pl.program_id(0) == 0)
    def _():
        stats_ref[...] = blockstats

    @pl.when(pl.program_id(0) != 0)
    def _():
        stats_ref[...] = stats_ref[...] + blockstats


def _layer_post(agg, dis, st, hin):
    nb = 1000
    return pl.pallas_call(
        _post_kernel,
        grid=(_N // nb,),
        in_specs=[
            pl.BlockSpec((2, nb, _H), lambda i: (0, i, 0)),
            pl.BlockSpec((nb, 1), lambda i: (i, 0)),
            pl.BlockSpec((nb, _EMB), lambda i: (i, 0)),
            pl.BlockSpec((nb, _EMB), lambda i: (i, 0)),
        ],
        out_specs=[
            pl.BlockSpec((nb, _EMB), lambda i: (i, 0)),
            pl.BlockSpec((2, _EMB), lambda i: (0, 0)),
        ],
        out_shape=[_f32(_N, _EMB), _f32(2, _EMB)],
    )(agg, dis, st, hin)


def _bn_from_stats(z, stats_ref, g_ref, b_ref, relu):
    mu = stats_ref[0:1, :] * (1.0 / _N)
    var = stats_ref[1:2, :] * (1.0 / _N) - mu * mu
    inv = lax.rsqrt(var + 1e-5)
    h = (z - mu) * inv * g_ref[...] + b_ref[...]
    if relu:
        h = jnp.maximum(h, 0.0)
    return h


def _bnpre_kernel(z_ref, stats_ref, bng_ref, bnb_ref, dis_ref, r_ref,
                  w_ref, b_ref, root_ref, g2_ref, st_ref, h_ref):
    h = _bn_from_stats(z_ref[...], stats_ref, bng_ref, bnb_ref, relu=True)
    h_ref[...] = h
    hp = jnp.dot(h, w_ref[...].T, preferred_element_type=_F32) + b_ref[...]
    g = hp * dis_ref[...]
    g2_ref[0] = g[:, :_H]
    g2_ref[1] = g[:, _H:]
    st_ref[...] = jnp.maximum(hp + root_ref[...], 0.0) * r_ref[...]


def _bn_layer_pre(z, stats, bng, bnb, dis, r, w, b, root):
    nb = 1000
    return pl.pallas_call(
        _bnpre_kernel,
        grid=(_N // nb,),
        in_specs=[
            pl.BlockSpec((nb, _EMB), lambda i: (i, 0)),
            pl.BlockSpec((2, _EMB), lambda i: (0, 0)),
            pl.BlockSpec((1, _EMB), lambda i: (0, 0)),
            pl.BlockSpec((1, _EMB), lambda i: (0, 0)),
            pl.BlockSpec((nb, 1), lambda i: (i, 0)),
            pl.BlockSpec((nb, 1), lambda i: (i, 0)),
            pl.BlockSpec((_EMB, _EMB), lambda i: (0, 0)),
            pl.BlockSpec((1, _EMB), lambda i: (0, 0)),
            pl.BlockSpec((1, _EMB), lambda i: (0, 0)),
        ],
        out_specs=[
            pl.BlockSpec((2, nb, _H), lambda i: (0, i, 0)),
            pl.BlockSpec((nb, _EMB), lambda i: (i, 0)),
            pl.BlockSpec((nb, _EMB), lambda i: (i, 0)),
        ],
        out_shape=[_f32(2, _N, _H), _f32(_N, _EMB), _f32(_N, _EMB)],
    )(z, stats, bng.reshape(1, _EMB), bnb.reshape(1, _EMB), dis, r, w,
      b.reshape(1, _EMB), root)


def _pool_kernel(z_ref, stats_ref, bng_ref, bnb_ref, b3_ref, sums_ref,
                 cnt_ref):
    h = _bn_from_stats(z_ref[...], stats_ref, bng_ref, bnb_ref, relu=False)
    ids = b3_ref[0, 0, :]
    io = lax.broadcasted_iota(jnp.int32, (_G, ids.shape[0]), 0)
    oh = (io == ids[None, :]).astype(_F32)
    ps = jnp.dot(oh, h, preferred_element_type=_F32)
    pc = jnp.sum(oh, axis=1, keepdims=True)

    @pl.when(pl.program_id(0) == 0)
    def _():
        sums_ref[...] = ps
        cnt_ref[...] = pc

    @pl.when(pl.program_id(0) != 0)
    def _():
        sums_ref[...] = sums_ref[...] + ps
        cnt_ref[...] = cnt_ref[...] + pc


def _pool(z, stats, bng, bnb, batch):
    nb = 1000
    batch3 = batch.reshape(_N // nb, 1, nb)
    return pl.pallas_call(
        _pool_kernel,
        grid=(_N // nb,),
        in_specs=[
            pl.BlockSpec((nb, _EMB), lambda i: (i, 0)),
            pl.BlockSpec((2, _EMB), lambda i: (0, 0)),
            pl.BlockSpec((1, _EMB), lambda i: (0, 0)),
            pl.BlockSpec((1, _EMB), lambda i: (0, 0)),
            pl.BlockSpec((1, 1, nb), lambda i: (i, 0, 0)),
        ],
        out_specs=[
            pl.BlockSpec((_G, _EMB), lambda i: (0, 0)),
            pl.BlockSpec((_G, 1), lambda i: (0, 0)),
        ],
        out_shape=[_f32(_G, _EMB), _f32(_G, 1)],
    )(z, stats, bng.reshape(1, _EMB), bnb.reshape(1, _EMB), batch3)


def _mlp_kernel(s_ref, c_ref, w1_ref, b1_ref, w2_ref, b2_ref, w3_ref, b3_ref,
                o_ref):
    hg = s_ref[...] / jnp.maximum(c_ref[...], 1.0)
    z = jnp.maximum(
        jnp.dot(hg, w1_ref[...].T, preferred_element_type=_F32) + b1_ref[...],
        0.0)
    z = jnp.maximum(
        jnp.dot(z, w2_ref[...].T, preferred_element_type=_F32) + b2_ref[...],
        0.0)
    o_ref[...] = (
        jnp.sum(z * w3_ref[...], axis=1, keepdims=True) + b3_ref[...]
    )


def _mlp(sums, cnt, pred):
    (w1, b1), (w2, b2), (w3, b3) = pred
    hh = w1.shape[0]
    return pl.pallas_call(
        _mlp_kernel,
        out_shape=_f32(_G, 1),
    )(sums, cnt, w1, b1.reshape(1, hh), w2, b2.reshape(1, hh),
      w3, b3.reshape(1, 1))


def _sc_mesh():
    return plsc.VectorSubcoreMesh(core_axis_name="c", subcore_axis_name="s")


def _sc_params():
    return pltpu.CompilerParams(use_tc_tiling_on_sc=False)


def _acc_init(zeros_hbm, acc_sh, s):
    @pl.when(s < _NS - 1)
    def _():
        pltpu.sync_copy(zeros_hbm, acc_sh.at[pl.ds(s * _ZB, _ZB)])

    @pl.when(s == _NS - 1)
    def _():
        pltpu.sync_copy(zeros_hbm.at[pl.ds(0, _ZT)],
                        acc_sh.at[pl.ds((_NS - 1) * _ZB, _ZT)])


def _acc_flush(acc_sh, out_c, s):
    @pl.when(s < _NS - 1)
    def _():
        pltpu.sync_copy(acc_sh.at[pl.ds(s * _ZB, _ZB)],
                        out_c.at[pl.ds(s * _ZB, _ZB)])

    @pl.when(s == _NS - 1)
    def _():
        pltpu.sync_copy(acc_sh.at[pl.ds((_NS - 1) * _ZB, _ZT)],
                        out_c.at[pl.ds((_NS - 1) * _ZB, _ZT)])


def _deg_sc_body(row_hbm, ones_hbm, zeros_hbm, out_hbm, idx0, idx1, ones_v,
                 semi0, semi1, sems0, sems1, acc_sh):
    c = lax.axis_index("c")
    s = lax.axis_index("s")
    _acc_init(zeros_hbm, acc_sh, s)
    pltpu.sync_copy(ones_hbm, ones_v)
    plsc.subcore_barrier()
    my_n = 195 + (s < 5).astype(jnp.int32)
    my_base = c * (_NCHUNK // _NC) + s * 195 + jnp.minimum(s, 5)

    idx = (idx0, idx1)
    semi = (semi0, semi1)
    sems = (sems0, sems1)

    def idx_copy(ch, b):
        return pltpu.make_async_copy(row_hbm.at[pl.ds(ch * _CH, _CH)],
                                     idx[b], semi[b])

    def scatter_copy(b):
        return pltpu.make_async_copy(ones_v, acc_sh.at[idx[b]], sems[b])

    idx_copy(my_base, 0).start()
    idx_copy(my_base, 0).wait()

    @pl.loop(0, 98)
    def _(kk):
        for b in (0, 1):
            k = kk * 2 + b
            ch = my_base + k

            @pl.when(k < my_n)
            def _():
                @pl.when(k >= 1)
                def _():
                    scatter_copy(1 - b).wait()

                @pl.when(k + 1 < my_n)
                def _():
                    idx_copy(ch + 1, 1 - b).start()

                scatter_copy(b).start(add=True)

                @pl.when(k + 1 < my_n)
                def _():
                    idx_copy(ch + 1, 1 - b).wait()

    @pl.when(my_n == 196)
    def _():
        scatter_copy(1).wait()

    @pl.when(my_n == 195)
    def _():
        scatter_copy(0).wait()

    plsc.subcore_barrier()
    _acc_flush(acc_sh, out_hbm.at[c], s)


def _deg_sc(row, ones8, zeros8):
    kfn = pl.kernel(
        _deg_sc_body,
        out_type=_f32(_NC, _N, 8),
        mesh=_sc_mesh(),
        compiler_params=_sc_params(),
        scratch_types=[
            pltpu.VMEM((_CH,), jnp.int32),
            pltpu.VMEM((_CH,), jnp.int32),
            pltpu.VMEM((_CH, 8), _F32),
            pltpu.SemaphoreType.DMA,
            pltpu.SemaphoreType.DMA,
            pltpu.SemaphoreType.DMA,
            pltpu.SemaphoreType.DMA,
            pltpu.VMEM_SHARED((_N, 8), _F32),
        ],
    )
    return kfn(row, ones8, zeros8)


def _edge_split(s):
    my_n = 390 + (s < 10).astype(jnp.int32)
    my_base = s * 390 + jnp.minimum(s, 10)
    return my_n, my_base


def _prep_sc_body(disw_hbm, ea_hbm, row_hbm, out_hbm,
                  idx0, idx1, idx2, d0, d1, d2, e0, e1, e2,
                  semi0, semi1, semi2, semg0, semg1, semg2,
                  semo0, semo1, semo2):
    c = lax.axis_index("c")
    s = lax.axis_index("s")
    my_n, my_base = _edge_split(s)

    idx = (idx0, idx1, idx2)
    dv = (d0, d1, d2)
    ev = (e0, e1, e2)
    semi = (semi0, semi1, semi2)
    semg = (semg0, semg1, semg2)
    semo = (semo0, semo1, semo2)

    def in_copies(ch, b):
        return (
            pltpu.make_async_copy(row_hbm.at[pl.ds(ch * _CH, _CH)],
                                  idx[b], semi[b]),
            pltpu.make_async_copy(ea_hbm.at[c].at[pl.ds(ch * _CHP, _CHP)],
                                  ev[b], semi[b]),
        )

    def gather_copy(b):
        return pltpu.make_async_copy(disw_hbm.at[idx[b]], dv[b], semg[b])

    def out_copy(ch, b):
        off = pl.ds(ch * _CHP, _CHP)
        return pltpu.make_async_copy(ev[b], out_hbm.at[c].at[off], semo[b])

    def compute(b):
        @plsc.parallel_loop(0, _CHP, unroll=4)
        def _(rrow):
            for q in range(4):
                for j in range(0, _H, 16):
                    sle = (rrow, pl.ds(q * _H + j, 16))
                    sld = (rrow * 4 + q, pl.ds(j, 16))
                    ev[b][sle] = dv[b][sld] * ev[b][sle]

    for cp in in_copies(my_base, 0):
        cp.start()
    for cp in in_copies(my_base + 1, 1):
        cp.start()
    for cp in in_copies(my_base, 0):
        cp.wait()
    gather_copy(0).start()

    @pl.loop(0, 131)
    def _(kk):
        for d in (0, 1, 2):
            k = kk * 3 + d
            b = d
            ch = my_base + k
            bn = (d + 1) % 3
            bp = (d + 2) % 3

            @pl.when(k < my_n)
            def _():
                gather_copy(b).wait()

                @pl.when(k + 1 < my_n)
                def _():
                    for cp in in_copies(ch + 1, bn):
                        cp.wait()
                    gather_copy(bn).start()

                @pl.when(k >= 1)
                def _():
                    out_copy(ch - 1, bp).wait()

                @pl.when(k + 2 < my_n)
                def _():
                    for cp in in_copies(ch + 2, bp):
                        cp.start()

                compute(b)
                out_copy(ch, b).start()

    @pl.when(my_n == 391)
    def _():
        out_copy(my_base + 390, 390 % 3).wait()

    @pl.when(my_n == 390)
    def _():
        out_copy(my_base + 389, 389 % 3).wait()


def _prep_sc(disw, ea2, row):
    kfn = pl.kernel(
        _prep_sc_body,
        out_type=_f32(_NC, _E // 4, 128),
        mesh=_sc_mesh(),
        compiler_params=_sc_params(),
        scratch_types=[
            pltpu.VMEM((_CH,), jnp.int32),
            pltpu.VMEM((_CH,), jnp.int32),
            pltpu.VMEM((_CH,), jnp.int32),
            pltpu.VMEM((_CH, _H), _F32),
            pltpu.VMEM((_CH, _H), _F32),
            pltpu.VMEM((_CH, _H), _F32),
            pltpu.VMEM((_CHP, 128), _F32),
            pltpu.VMEM((_CHP, 128), _F32),
            pltpu.VMEM((_CHP, 128), _F32),
            pltpu.SemaphoreType.DMA,
            pltpu.SemaphoreType.DMA,
            pltpu.SemaphoreType.DMA,
            pltpu.SemaphoreType.DMA,
            pltpu.SemaphoreType.DMA,
            pltpu.SemaphoreType.DMA,
            pltpu.SemaphoreType.DMA,
            pltpu.SemaphoreType.DMA,
            pltpu.SemaphoreType.DMA,
        ],
    )
    return kfn(disw, ea2, row)


def _msg_sc_body(g2_hbm, eaw_hbm, row_hbm, col_hbm, zeros_hbm, out_hbm,
                 idxr0, idxr1, idxr2, idxc0, idxc1, idxc2,
                 g0, g1, g2v, e0, e1, e2,
                 semi0, semi1, semi2, semg0, semg1, semg2,
                 sems0, sems1, sems2, acc_sh):
    c = lax.axis_index("c")
    s = lax.axis_index("s")
    _acc_init(zeros_hbm, acc_sh, s)
    plsc.subcore_barrier()
    my_n, my_base = _edge_split(s)

    idxr = (idxr0, idxr1, idxr2)
    idxc = (idxc0, idxc1, idxc2)
    gv = (g0, g1, g2v)
    ev = (e0, e1, e2)
    semi = (semi0, semi1, semi2)
    semg = (semg0, semg1, semg2)
    sems = (sems0, sems1, sems2)

    def in_copies(ch, b):
        off = pl.ds(ch * _CH, _CH)
        return (
            pltpu.make_async_copy(row_hbm.at[off], idxr[b], semi[b]),
            pltpu.make_async_copy(col_hbm.at[off], idxc[b], semi[b]),
            pltpu.make_async_copy(eaw_hbm.at[c].at[pl.ds(ch * _CHP, _CHP)],
                                  ev[b], semi[b]),
        )

    def issue_inputs(ch, b):
        for cp in in_copies(ch, b):
            cp.start()

    def wait_inputs(ch, b):
        for cp in in_copies(ch, b):
            cp.wait()

    def gather_copy(b):
        return pltpu.make_async_copy(g2_hbm.at[c].at[idxr[b]], gv[b], semg[b])

    def scatter_copy(b):
        return pltpu.make_async_copy(gv[b], acc_sh.at[idxc[b]], sems[b])

    def compute(b):
        @plsc.parallel_loop(0, _CHP, unroll=4)
        def _(rrow):
            for q in range(4):
                for j in range(0, _H, 16):
                    slg = (rrow * 4 + q, pl.ds(j, 16))
                    sle = (rrow, pl.ds(q * _H + j, 16))
                    gv[b][slg] = jnp.maximum(gv[b][slg] + ev[b][sle], 0.0)

    issue_inputs(my_base, 0)
    issue_inputs(my_base + 1, 1)
    wait_inputs(my_base, 0)
    gather_copy(0).start()

    @pl.loop(0, 131)
    def _(kk):
        for d in (0, 1, 2):
            k = kk * 3 + d
            b = d
            ch = my_base + k
            bn = (d + 1) % 3
            bp = (d + 2) % 3

            @pl.when(k < my_n)
            def _():
                gather_copy(b).wait()

                @pl.when(k + 1 < my_n)
                def _():
                    wait_inputs(ch + 1, bn)
                    gather_copy(bn).start()

                @pl.when(k >= 1)
                def _():
                    scatter_copy(bp).wait()

                @pl.when(k + 2 < my_n)
                def _():
                    issue_inputs(ch + 2, bp)

                compute(b)
                scatter_copy(b).start(add=True)

    @pl.when(my_n == 391)
    def _():
        scatter_copy(390 % 3).wait()

    @pl.when(my_n == 390)
    def _():
        scatter_copy(389 % 3).wait()

    plsc.subcore_barrier()
    _acc_flush(acc_sh, out_hbm.at[c], s)


def _msg_sc(g2, eaw2, row, col, zeros_h):
    kfn = pl.kernel(
        _msg_sc_body,
        out_type=_f32(_NC, _N, _H),
        mesh=_sc_mesh(),
        compiler_params=_sc_params(),
        scratch_types=[
            pltpu.VMEM((_CH,), jnp.int32),
            pltpu.VMEM((_CH,), jnp.int32),
            pltpu.VMEM((_CH,), jnp.int32),
            pltpu.VMEM((_CH,), jnp.int32),
            pltpu.VMEM((_CH,), jnp.int32),
            pltpu.VMEM((_CH,), jnp.int32),
            pltpu.VMEM((_CH, _H), _F32),
            pltpu.VMEM((_CH, _H), _F32),
            pltpu.VMEM((_CH, _H), _F32),
            pltpu.VMEM((_CHP, 128), _F32),
            pltpu.VMEM((_CHP, 128), _F32),
            pltpu.VMEM((_CHP, 128), _F32),
            pltpu.SemaphoreType.DMA,
            pltpu.SemaphoreType.DMA,
            pltpu.SemaphoreType.DMA,
            pltpu.SemaphoreType.DMA,
            pltpu.SemaphoreType.DMA,
            pltpu.SemaphoreType.DMA,
            pltpu.SemaphoreType.DMA,
            pltpu.SemaphoreType.DMA,
            pltpu.SemaphoreType.DMA,
            pltpu.VMEM_SHARED((_N, _H), _F32),
        ],
    )
    return kfn(g2, eaw2, row, col, zeros_h)


def kernel(x, edge_index, edge_attr, batch, params):
    row = edge_index[0]
    col = edge_index[1]

    ones8 = jnp.ones((_CH, 8), _F32)
    zeros8 = jnp.zeros((_ZB, 8), _F32)
    zeros_h = jnp.zeros((_ZB, _H), _F32)

    h = _node_embed(x, params['x_emb_W'], params['x_emb_b'])
    ea2 = _edge_embed(edge_attr, params['edge_emb_W'], params['edge_emb_b'])

    degpart = _deg_sc(row, ones8, zeros8)
    dis, r, disw = _degpost(degpart)

    eaw2 = _prep_sc(disw, ea2, row)

    z = stats = None
    for l in range(_NL):
        lp = params['layers'][l]
        if l == 0:
            g2, st = _layer_pre(h, dis, r, lp['lin_W'], lp['lin_b'],
                                lp['root'])
        else:
            bp = params['layers'][l - 1]
            g2, st, h = _bn_layer_pre(z, stats, bp['bn_g'], bp['bn_b'],
                                      dis, r, lp['lin_W'], lp['lin_b'],
                                      lp['root'])
        agg = _msg_sc(g2, eaw2, row, col, zeros_h)
        z, stats = _layer_post(agg, dis, st, h)

    lp = params['layers'][_NL - 1]
    sums, cnt = _pool(z, stats, lp['bn_g'], lp['bn_b'], batch)
    return _mlp(sums, cnt, params['pred'])
